# baseline (speedup 1.0000x reference)
import functools

import jax
import jax.numpy as jnp
from jax import lax
from jax.experimental import pallas as pl
from jax.experimental.pallas import tpu as pltpu
from jax.experimental.pallas import tpu_sc as plsc

EPS = 1e-5
H = 128
NC, NS, LANES = 2, 16, 16
NW = NC * NS
CH = 80
SUP = 1280
f32 = jnp.float32


def _sc_mesh():
    return plsc.VectorSubcoreMesh(core_axis_name="c", subcore_axis_name="s",
                                  num_cores=NC, num_subcores=NS)


_SC_PARAMS = pltpu.CompilerParams(use_tc_tiling_on_sc=False)


def _build_pass1(e_pad, e_valid):
    ept = e_pad // NW
    n_chunks = ept // CH

    def body(ps_hbm, pr_hbm, c_hbm, i0_hbm, i1_hbm, pre_hbm, st_hbm,
             i0_v, i1_v, ps_v, pr_v, c_v, st_v, s1, s2, s3):
        cid = lax.axis_index("c")
        sid = lax.axis_index("s")
        wid = sid * NC + cid
        chunk0 = wid * n_chunks
        for v in range(16):
            st_v[pl.ds(v * LANES, LANES)] = jnp.zeros((LANES,), f32)

        def chunk(i, _):
            ci = chunk0 + i
            e0 = ci * CH
            pltpu.sync_copy(i0_hbm.at[ci], i0_v)
            pltpu.sync_copy(i1_hbm.at[ci], i1_v)
            cp1 = pltpu.async_copy(ps_hbm.at[i0_v], ps_v, s1)
            cp2 = pltpu.async_copy(pr_hbm.at[i1_v], pr_v, s2)
            cp3 = pltpu.async_copy(c_hbm.at[pl.ds(e0, CH), :], c_v, s3)
            cp1.wait()
            cp2.wait()
            cp3.wait()

            def row(j, carry):
                acc = list(carry)
                for v in range(8):
                    sl = pl.ds(v * LANES, LANES)
                    p = ps_v[j, sl] + pr_v[j, sl] + c_v[j, sl]
                    ps_v[j, sl] = p
                    acc[v] = acc[v] + p
                    acc[8 + v] = acc[8 + v] + p * p
                return tuple(acc)

            zero16 = tuple(jnp.zeros((LANES,), f32) for _ in range(16))
            acc = lax.fori_loop(0, CH, row, zero16)

            @pl.when(e0 < e_valid)
            def _():
                for v in range(16):
                    sl = pl.ds(v * LANES, LANES)
                    st_v[sl] = st_v[sl] + acc[v]

            pltpu.sync_copy(ps_v, pre_hbm.at[pl.ds(e0, CH), :])
            return 0

        lax.fori_loop(0, n_chunks, chunk, 0)
        pltpu.sync_copy(st_v, st_hbm.at[wid])

    return pl.kernel(
        body,
        out_type=[jax.ShapeDtypeStruct((e_pad, H), f32),
                  jax.ShapeDtypeStruct((NW, 2 * H), f32)],
        mesh=_sc_mesh(),
        scratch_types=[
            pltpu.VMEM((CH,), jnp.int32), pltpu.VMEM((CH,), jnp.int32),
            pltpu.VMEM((CH, H), f32), pltpu.VMEM((CH, H), f32),
            pltpu.VMEM((CH, H), f32), pltpu.VMEM((2 * H,), f32),
            pltpu.SemaphoreType.DMA, pltpu.SemaphoreType.DMA,
            pltpu.SemaphoreType.DMA,
        ],
        compiler_params=_SC_PARAMS)


def _build_pass3(e_pad, n_pad):
    ept = e_pad // NS
    n_sup, tail = divmod(ept, SUP)
    rpt = n_pad // NS
    zr = rpt // 8

    def body(val_hbm, i1_hbm, mes_hbm, idx_sb, val_sb, zero_v, acc_sh, sem):
        cid = lax.axis_index("c")
        sid = lax.axis_index("s")
        base = sid * ept
        cbase = base // CH
        r0 = sid * rpt

        def zrow(j, _):
            zero_v[j] = jnp.zeros((LANES,), f32)
            return 0
        lax.fori_loop(0, zr, zrow, 0)
        for z in range(8):
            pltpu.sync_copy(zero_v, acc_sh.at[pl.ds(r0 + z * zr, zr)])
        plsc.subcore_barrier()

        for gl in range(4):
            g = cid * 4 + gl
            col = g * LANES

            def do_super(c0, e0, n):
                nch = n // CH
                pltpu.sync_copy(i1_hbm.at[pl.ds(c0, nch)],
                                idx_sb.at[pl.ds(0, nch)])
                cp = pltpu.async_copy(
                    val_hbm.at[pl.ds(e0, n), pl.ds(col, LANES)],
                    val_sb.at[pl.ds(0, n)], sem)
                cp.wait()
                for j in range(nch):
                    pltpu.sync_copy(val_sb.at[pl.ds(j * CH, CH)],
                                    acc_sh.at[idx_sb.at[j]], add=True)

            def sup_loop(s, _):
                do_super(cbase + s * (SUP // CH), base + s * SUP, SUP)
                return 0
            lax.fori_loop(0, n_sup, sup_loop, 0)
            if tail:
                do_super(cbase + n_sup * (SUP // CH), base + n_sup * SUP, tail)

            plsc.subcore_barrier()
            pltpu.sync_copy(acc_sh.at[pl.ds(r0, rpt)],
                            mes_hbm.at[pl.ds(r0, rpt), pl.ds(col, LANES)])
            for z in range(8):
                pltpu.sync_copy(zero_v, acc_sh.at[pl.ds(r0 + z * zr, zr)])
            plsc.subcore_barrier()

    return pl.kernel(
        body,
        out_type=jax.ShapeDtypeStruct((n_pad, H), f32),
        mesh=_sc_mesh(),
        scratch_types=[
            pltpu.VMEM((SUP // CH, CH), jnp.int32),
            pltpu.VMEM((SUP, LANES), f32),
            pltpu.VMEM((zr, LANES), f32),
            pltpu.VMEM_SHARED((n_pad, LANES), f32),
            pltpu.SemaphoreType.DMA,
        ],
        compiler_params=_SC_PARAMS)


def _proj3(x, wcat):
    n = x.shape[0]
    bn = 1000

    def body(x_ref, w_ref, o0, o1, o2):
        xb = x_ref[...]
        o0[...] = jnp.dot(xb, w_ref[:, 0:H], preferred_element_type=f32)
        o1[...] = jnp.dot(xb, w_ref[:, H:2 * H], preferred_element_type=f32)
        o2[...] = jnp.dot(xb, w_ref[:, 2 * H:3 * H], preferred_element_type=f32)

    return pl.pallas_call(
        body,
        out_shape=[jax.ShapeDtypeStruct((n, H), f32)] * 3,
        grid=(n // bn,),
        in_specs=[pl.BlockSpec((bn, H), lambda i: (i, 0)),
                  pl.BlockSpec((H, 3 * H), lambda i: (0, 0))],
        out_specs=[pl.BlockSpec((bn, H), lambda i: (i, 0))] * 3,
    )(x, wcat)


def _cmats(e_pad, invs, wvs, bms):
    be = 1280
    nadj = len(invs)
    ninvs = [v.shape[1] for v in invs]

    def body(*refs):
        in_refs = refs[:3 * nadj]
        out_refs = refs[3 * nadj:]
        for a in range(nadj):
            inv_ref, wv_ref, bm_ref = in_refs[a], in_refs[nadj + a], in_refs[2 * nadj + a]
            acc = jnp.broadcast_to(bm_ref[...], (be, H))
            for j in range(ninvs[a]):
                acc = acc + inv_ref[:, j:j + 1] * wv_ref[j:j + 1, :]
            out_refs[a][...] = acc

    return pl.pallas_call(
        body,
        out_shape=[jax.ShapeDtypeStruct((e_pad, H), f32)] * nadj,
        grid=(e_pad // be,),
        in_specs=(
            [pl.BlockSpec((be, ninvs[a]), lambda i: (i, 0)) for a in range(nadj)]
            + [pl.BlockSpec((8, H), lambda i: (0, 0))] * nadj
            + [pl.BlockSpec((1, H), lambda i: (0, 0))] * nadj),
        out_specs=[pl.BlockSpec((be, H), lambda i: (i, 0))] * nadj,
    )(*invs, *wvs, *bms)


def _coefs(e_valid, stats, gms, bbs, bis):
    nadj = len(stats)

    def body(*refs):
        st = refs[:nadj]
        gm = refs[nadj:2 * nadj]
        bb = refs[2 * nadj:3 * nadj]
        bi = refs[3 * nadj:4 * nadj]
        outs = refs[4 * nadj:]
        for a in range(nadj):
            s = st[a][...]
            mean = jnp.sum(s[:, :H], axis=0, keepdims=True) / e_valid
            var = jnp.sum(s[:, H:], axis=0, keepdims=True) / e_valid - mean * mean
            scale = gm[a][...] * lax.rsqrt(var + EPS)
            shift = bb[a][...] - mean * scale
            outs[a][...] = jnp.concatenate(
                [scale, shift, bi[a][...], jnp.zeros((5, H), f32)], axis=0)

    return pl.pallas_call(
        body,
        out_shape=[jax.ShapeDtypeStruct((8, H), f32)] * nadj,
        in_specs=[pl.BlockSpec((NW, 2 * H), lambda: (0, 0))] * nadj
        + [pl.BlockSpec((1, H), lambda: (0, 0))] * (3 * nadj),
        out_specs=[pl.BlockSpec((8, H), lambda: (0, 0))] * nadj,
    )(*stats, *gms, *bbs, *bis)


def _val(e_pad, e_valid, pre, coef, wi):
    br = 128

    def body(pre_ref, coef_ref, wi_ref, o_ref):
        i = pl.program_id(0)
        scale = coef_ref[0:1, :]
        shift = coef_ref[1:2, :]
        t = pre_ref[...] * scale + shift
        msg = t * jax.nn.sigmoid(t)
        z = jnp.dot(msg, wi_ref[...], preferred_element_type=f32)
        w = jax.nn.sigmoid(z + coef_ref[2:3, 0:1])
        rows = i * br + lax.broadcasted_iota(jnp.int32, (br, 1), 0)
        w = jnp.where(rows < e_valid, w, 0.0)
        o_ref[...] = msg * w

    return pl.pallas_call(
        body,
        out_shape=jax.ShapeDtypeStruct((e_pad, H), f32),
        grid=(e_pad // br,),
        in_specs=[pl.BlockSpec((br, H), lambda i: (i, 0)),
                  pl.BlockSpec((8, H), lambda i: (0, 0)),
                  pl.BlockSpec((H, 1), lambda i: (0, 0))],
        out_specs=pl.BlockSpec((br, H), lambda i: (i, 0)),
    )(pre, coef, wi)


def _update(x, mes_list, wu_parts, bu, gu, bbu):
    n = x.shape[0]
    bn = 1000
    steps = n // bn
    nm = len(mes_list)

    def ubody(refs):
        x_ref = refs[0]
        m_refs = refs[1:1 + nm]
        w_refs = refs[1 + nm:2 + 2 * nm]
        bu_ref = refs[2 + 2 * nm]
        u = jnp.dot(x_ref[...], w_refs[0][...], preferred_element_type=f32)
        for k in range(nm):
            u = u + jnp.dot(m_refs[k][...], w_refs[1 + k][...],
                            preferred_element_type=f32)
        return u + bu_ref[...]

    def body_a(*refs):
        i = pl.program_id(0)
        o_ref, acc_ref = refs[-2], refs[-1]
        u = ubody(refs[:-2])

        @pl.when(i == 0)
        def _():
            acc_ref[...] = jnp.zeros((8, H), f32)

        acc_ref[0:1, :] += jnp.sum(u, axis=0, keepdims=True)
        acc_ref[1:2, :] += jnp.sum(u * u, axis=0, keepdims=True)

        @pl.when(i == steps - 1)
        def _():
            o_ref[...] = acc_ref[...]

    def body_b(*refs):
        gu_ref, bbu_ref, st_ref, o_ref = refs[-4:]
        u = ubody(refs[:-4])
        mean = st_ref[0:1, :] / n
        var = st_ref[1:2, :] / n - mean * mean
        scale = gu_ref[...] * lax.rsqrt(var + EPS)
        shift = bbu_ref[...] - mean * scale
        o_ref[...] = refs[0][...] + u * scale + shift

    data_specs = ([pl.BlockSpec((bn, H), lambda i: (i, 0))] * (1 + nm)
                  + [pl.BlockSpec((H, H), lambda i: (0, 0))] * (1 + nm)
                  + [pl.BlockSpec((1, H), lambda i: (0, 0))])
    stats = pl.pallas_call(
        body_a,
        out_shape=jax.ShapeDtypeStruct((8, H), f32),
        grid=(steps,),
        in_specs=data_specs,
        out_specs=pl.BlockSpec((8, H), lambda i: (0, 0)),
        scratch_shapes=[pltpu.VMEM((8, H), f32)],
    )(x, *mes_list, *wu_parts, bu)

    return pl.pallas_call(
        body_b,
        out_shape=jax.ShapeDtypeStruct((n, H), f32),
        grid=(steps,),
        in_specs=data_specs + [pl.BlockSpec((1, H), lambda i: (0, 0))] * 2
        + [pl.BlockSpec((8, H), lambda i: (0, 0))],
        out_specs=pl.BlockSpec((bn, H), lambda i: (i, 0)),
    )(x, *mes_list, *wu_parts, bu, gu, bbu, stats)


def _pad_idx(idx, e_pad):
    e = idx.shape[0]
    pad = jnp.arange(e_pad - e, dtype=jnp.int32) % 256
    return jnp.concatenate([idx, pad]).reshape(e_pad // CH, CH)


def _pad_inv(inv, e_pad):
    e = inv.shape[0]
    return jnp.concatenate(
        [inv, jnp.zeros((e_pad - e, inv.shape[1]), inv.dtype)])


def _row(v):
    return v.reshape(1, -1)


def kernel(x_0, x_1, adj_0_0, adj_0_1, adj_1_1, inv_0_0, inv_0_1, inv_1_1,
           Wm_00, bm_00, gm_00, bb_00, Wi_00, bi_00,
           Wm_01, bm_01, gm_01, bb_01, Wi_01, bi_01,
           Wm_11, bm_11, gm_11, bb_11, Wi_11, bi_11,
           Wu_0, bu_0, gu_0, bbu_0, Wu_1, bu_1, gu_1, bbu_1):
    n0, n1 = x_0.shape[0], x_1.shape[0]
    e = adj_0_0.shape[1]
    assert e % CH == 0
    n_chunks = -(-e // CH)
    e_pad = -(-n_chunks // NW) * NW * CH
    np0 = -(-n0 // 2048) * 2048
    np1 = -(-n1 // 2048) * 2048

    ps00, pr00, ps01 = _proj3(x_0, jnp.concatenate(
        [Wm_00[:H], Wm_00[H:2 * H], Wm_01[:H]], axis=1))
    pr01, ps11, pr11 = _proj3(x_1, jnp.concatenate(
        [Wm_01[H:2 * H], Wm_11[:H], Wm_11[H:2 * H]], axis=1))

    def pad_w(w):
        return jnp.concatenate([w, jnp.zeros((8 - w.shape[0], H), f32)])
    c00, c01, c11 = _cmats(
        e_pad,
        [_pad_inv(inv_0_0, e_pad), _pad_inv(inv_0_1, e_pad), _pad_inv(inv_1_1, e_pad)],
        [pad_w(Wm_00[2 * H:]), pad_w(Wm_01[2 * H:]), pad_w(Wm_11[2 * H:])],
        [_row(bm_00), _row(bm_01), _row(bm_11)])

    idx = {
        "00": (_pad_idx(adj_0_0[0], e_pad), _pad_idx(adj_0_0[1], e_pad)),
        "01": (_pad_idx(adj_0_1[0], e_pad), _pad_idx(adj_0_1[1], e_pad)),
        "11": (_pad_idx(adj_1_1[0], e_pad), _pad_idx(adj_1_1[1], e_pad)),
    }

    p1 = _build_pass1(e_pad, e)
    pre00, st00 = p1(ps00, pr00, c00, idx["00"][0], idx["00"][1])
    pre01, st01 = p1(ps01, pr01, c01, idx["01"][0], idx["01"][1])
    pre11, st11 = p1(ps11, pr11, c11, idx["11"][0], idx["11"][1])

    cf00, cf01, cf11 = _coefs(
        float(e), [st00, st01, st11],
        [_row(gm_00), _row(gm_01), _row(gm_11)],
        [_row(bb_00), _row(bb_01), _row(bb_11)],
        [jnp.broadcast_to(bi_00, (1, H)), jnp.broadcast_to(bi_01, (1, H)),
         jnp.broadcast_to(bi_11, (1, H))])

    val00 = _val(e_pad, e, pre00, cf00, Wi_00)
    val01 = _val(e_pad, e, pre01, cf01, Wi_01)
    val11 = _val(e_pad, e, pre11, cf11, Wi_11)

    p3_0 = _build_pass3(e_pad, np0)
    p3_1 = _build_pass3(e_pad, np1)
    mes00 = p3_0(val00, idx["00"][1])
    mes01 = p3_1(val01, idx["01"][1])
    mes11 = p3_1(val11, idx["11"][1])

    out0 = _update(x_0, [mes00], [Wu_0[:H], Wu_0[H:]],
                   _row(bu_0), _row(gu_0), _row(bbu_0))
    out1 = _update(x_1, [mes01, mes11], [Wu_1[:H], Wu_1[H:2 * H], Wu_1[2 * H:]],
                   _row(bu_1), _row(gu_1), _row(bbu_1))
    return (out0, out1)


# speedup vs baseline: 1.9396x; 1.9396x over previous
import functools

import jax
import jax.numpy as jnp
from jax import lax
from jax.experimental import pallas as pl
from jax.experimental.pallas import tpu as pltpu
from jax.experimental.pallas import tpu_sc as plsc

EPS = 1e-5
H = 128
NC, NS, LANES = 2, 16, 16
NW = NC * NS
CH = 80
SUP = 1280
f32 = jnp.float32


def _sc_mesh():
    return plsc.VectorSubcoreMesh(core_axis_name="c", subcore_axis_name="s",
                                  num_cores=NC, num_subcores=NS)


_SC_PARAMS = pltpu.CompilerParams(use_tc_tiling_on_sc=False)


def _build_pass1(e_pad, e_valid):
    ept = e_pad // NW
    n_chunks = ept // CH

    def body(ps_hbm, pr_hbm, c_hbm, i0_hbm, i1_hbm, pre_hbm, st_hbm,
             i0_v, i1_v, ps_v, pr_v, c_v, st_v, s1, s2, s3):
        cid = lax.axis_index("c")
        sid = lax.axis_index("s")
        wid = sid * NC + cid
        chunk0 = wid * n_chunks
        for v in range(16):
            st_v[pl.ds(v * LANES, LANES)] = jnp.zeros((LANES,), f32)

        def chunk(i, _):
            ci = chunk0 + i
            e0 = ci * CH
            pltpu.sync_copy(i0_hbm.at[ci], i0_v)
            pltpu.sync_copy(i1_hbm.at[ci], i1_v)
            cp1 = pltpu.async_copy(ps_hbm.at[i0_v], ps_v, s1)
            cp2 = pltpu.async_copy(pr_hbm.at[i1_v], pr_v, s2)
            cp3 = pltpu.async_copy(c_hbm.at[pl.ds(e0, CH), :], c_v, s3)
            cp1.wait()
            cp2.wait()
            cp3.wait()

            def row(j, carry):
                acc = list(carry)
                for v in range(8):
                    sl = pl.ds(v * LANES, LANES)
                    p = ps_v[j, sl] + pr_v[j, sl] + c_v[j, sl]
                    ps_v[j, sl] = p
                    acc[v] = acc[v] + p
                    acc[8 + v] = acc[8 + v] + p * p
                return tuple(acc)

            zero16 = tuple(jnp.zeros((LANES,), f32) for _ in range(16))
            acc = lax.fori_loop(0, CH, row, zero16)

            @pl.when(e0 < e_valid)
            def _():
                for v in range(16):
                    sl = pl.ds(v * LANES, LANES)
                    st_v[sl] = st_v[sl] + acc[v]

            pltpu.sync_copy(ps_v, pre_hbm.at[pl.ds(e0, CH), :])
            return 0

        lax.fori_loop(0, n_chunks, chunk, 0)
        pltpu.sync_copy(st_v, st_hbm.at[wid])

    return pl.kernel(
        body,
        out_type=[jax.ShapeDtypeStruct((e_pad, H), f32),
                  jax.ShapeDtypeStruct((NW, 2 * H), f32)],
        mesh=_sc_mesh(),
        scratch_types=[
            pltpu.VMEM((CH,), jnp.int32), pltpu.VMEM((CH,), jnp.int32),
            pltpu.VMEM((CH, H), f32), pltpu.VMEM((CH, H), f32),
            pltpu.VMEM((CH, H), f32), pltpu.VMEM((2 * H,), f32),
            pltpu.SemaphoreType.DMA, pltpu.SemaphoreType.DMA,
            pltpu.SemaphoreType.DMA,
        ],
        compiler_params=_SC_PARAMS)


def _build_pass3(e_pad, n_pad):
    ept = e_pad // NS
    n_sup, tail = divmod(ept, SUP)
    rpt = n_pad // NS
    zr = rpt // 8

    def body(val_hbm, i1_hbm, mes_hbm, idx_sb, val_sb, zero_v, acc_sh, sem):
        cid = lax.axis_index("c")
        sid = lax.axis_index("s")
        base = sid * ept
        cbase = base // CH
        r0 = sid * rpt

        def zrow(j, _):
            zero_v[j] = jnp.zeros((LANES,), f32)
            return 0
        lax.fori_loop(0, zr, zrow, 0)
        for z in range(8):
            pltpu.sync_copy(zero_v, acc_sh.at[pl.ds(r0 + z * zr, zr)])
        plsc.subcore_barrier()

        for gl in range(4):
            g = cid * 4 + gl
            col = g * LANES

            def do_super(c0, e0, n):
                nch = n // CH
                pltpu.sync_copy(i1_hbm.at[pl.ds(c0, nch)],
                                idx_sb.at[pl.ds(0, nch)])
                cp = pltpu.async_copy(
                    val_hbm.at[pl.ds(e0, n), pl.ds(col, LANES)],
                    val_sb.at[pl.ds(0, n)], sem)
                cp.wait()
                for j in range(nch):
                    pltpu.sync_copy(val_sb.at[pl.ds(j * CH, CH)],
                                    acc_sh.at[idx_sb.at[j]], add=True)

            def sup_loop(s, _):
                do_super(cbase + s * (SUP // CH), base + s * SUP, SUP)
                return 0
            lax.fori_loop(0, n_sup, sup_loop, 0)
            if tail:
                do_super(cbase + n_sup * (SUP // CH), base + n_sup * SUP, tail)

            plsc.subcore_barrier()
            pltpu.sync_copy(acc_sh.at[pl.ds(r0, rpt)],
                            mes_hbm.at[pl.ds(r0, rpt), pl.ds(col, LANES)])
            for z in range(8):
                pltpu.sync_copy(zero_v, acc_sh.at[pl.ds(r0 + z * zr, zr)])
            plsc.subcore_barrier()

    return pl.kernel(
        body,
        out_type=jax.ShapeDtypeStruct((n_pad, H), f32),
        mesh=_sc_mesh(),
        scratch_types=[
            pltpu.VMEM((SUP // CH, CH), jnp.int32),
            pltpu.VMEM((SUP, LANES), f32),
            pltpu.VMEM((zr, LANES), f32),
            pltpu.VMEM_SHARED((n_pad, LANES), f32),
            pltpu.SemaphoreType.DMA,
        ],
        compiler_params=_SC_PARAMS)


def _proj3(x, wcat):
    n = x.shape[0]
    bn = 1000

    def body(x_ref, w_ref, o0, o1, o2):
        xb = x_ref[...]
        o0[...] = jnp.dot(xb, w_ref[:, 0:H], preferred_element_type=f32)
        o1[...] = jnp.dot(xb, w_ref[:, H:2 * H], preferred_element_type=f32)
        o2[...] = jnp.dot(xb, w_ref[:, 2 * H:3 * H], preferred_element_type=f32)

    return pl.pallas_call(
        body,
        out_shape=[jax.ShapeDtypeStruct((n, H), f32)] * 3,
        grid=(n // bn,),
        in_specs=[pl.BlockSpec((bn, H), lambda i: (i, 0)),
                  pl.BlockSpec((H, 3 * H), lambda i: (0, 0))],
        out_specs=[pl.BlockSpec((bn, H), lambda i: (i, 0))] * 3,
    )(x, wcat)


def _cmat1(e_pad, inv, wv, bm):
    be = 1280
    ninv = inv.shape[1]

    def body(inv_ref, wv_ref, bm_ref, o_ref):
        acc = jnp.broadcast_to(bm_ref[...], (be, H))
        for j in range(ninv):
            acc = acc + inv_ref[:, j:j + 1] * wv_ref[j:j + 1, :]
        o_ref[...] = acc

    return pl.pallas_call(
        body,
        out_shape=jax.ShapeDtypeStruct((e_pad, H), f32),
        grid=(e_pad // be,),
        in_specs=[pl.BlockSpec((be, ninv), lambda i: (i, 0)),
                  pl.BlockSpec((8, H), lambda i: (0, 0)),
                  pl.BlockSpec((1, H), lambda i: (0, 0))],
        out_specs=pl.BlockSpec((be, H), lambda i: (i, 0)),
    )(inv, wv, bm)


def _coef1(e_valid, st, gm, bb, bi):
    def body(st_ref, gm_ref, bb_ref, bi_ref, o_ref):
        s = st_ref[...]
        mean = jnp.sum(s[:, :H], axis=0, keepdims=True) / e_valid
        var = jnp.sum(s[:, H:], axis=0, keepdims=True) / e_valid - mean * mean
        scale = gm_ref[...] * lax.rsqrt(var + EPS)
        shift = bb_ref[...] - mean * scale
        o_ref[...] = jnp.concatenate(
            [scale, shift, bi_ref[...], jnp.zeros((5, H), f32)], axis=0)

    return pl.pallas_call(
        body,
        out_shape=jax.ShapeDtypeStruct((8, H), f32),
        in_specs=[pl.BlockSpec((NW, 2 * H), lambda: (0, 0))]
        + [pl.BlockSpec((1, H), lambda: (0, 0))] * 3,
        out_specs=pl.BlockSpec((8, H), lambda: (0, 0)),
    )(st, gm, bb, bi)


def _val(e_pad, e_valid, pre, coef, wi_pad):
    br = 512

    def body(pre_ref, coef_ref, wi_ref, o_ref):
        i = pl.program_id(0)
        scale = coef_ref[0:1, :]
        shift = coef_ref[1:2, :]
        t = pre_ref[...] * scale + shift
        msg = t * jax.nn.sigmoid(t)
        z = jnp.dot(msg, wi_ref[...], preferred_element_type=f32)
        w = jax.nn.sigmoid(z[:, 0:1] + coef_ref[2:3, 0:1])
        rows = i * br + lax.broadcasted_iota(jnp.int32, (br, 1), 0)
        w = jnp.where(rows < e_valid, w, 0.0)
        o_ref[...] = msg * w

    return pl.pallas_call(
        body,
        out_shape=jax.ShapeDtypeStruct((e_pad, H), f32),
        grid=(e_pad // br,),
        in_specs=[pl.BlockSpec((br, H), lambda i: (i, 0)),
                  pl.BlockSpec((8, H), lambda i: (0, 0)),
                  pl.BlockSpec((H, H), lambda i: (0, 0))],
        out_specs=pl.BlockSpec((br, H), lambda i: (i, 0)),
    )(pre, coef, wi_pad)


def _update(x, mes_list, wu_parts, bu, gu, bbu):
    n = x.shape[0]
    bn = 1000
    steps = n // bn
    nm = len(mes_list)

    def ubody(refs):
        x_ref = refs[0]
        m_refs = refs[1:1 + nm]
        w_refs = refs[1 + nm:2 + 2 * nm]
        bu_ref = refs[2 + 2 * nm]
        u = jnp.dot(x_ref[...], w_refs[0][...], preferred_element_type=f32)
        for k in range(nm):
            u = u + jnp.dot(m_refs[k][...], w_refs[1 + k][...],
                            preferred_element_type=f32)
        return u + bu_ref[...]

    def body_a(*refs):
        i = pl.program_id(0)
        o_ref, acc_ref = refs[-2], refs[-1]
        u = ubody(refs[:-2])

        @pl.when(i == 0)
        def _():
            acc_ref[...] = jnp.zeros((8, H), f32)

        acc_ref[0:1, :] += jnp.sum(u, axis=0, keepdims=True)
        acc_ref[1:2, :] += jnp.sum(u * u, axis=0, keepdims=True)

        @pl.when(i == steps - 1)
        def _():
            o_ref[...] = acc_ref[...]

    def body_b(*refs):
        gu_ref, bbu_ref, st_ref, o_ref = refs[-4:]
        u = ubody(refs[:-4])
        mean = st_ref[0:1, :] / n
        var = st_ref[1:2, :] / n - mean * mean
        scale = gu_ref[...] * lax.rsqrt(var + EPS)
        shift = bbu_ref[...] - mean * scale
        o_ref[...] = refs[0][...] + u * scale + shift

    data_specs = ([pl.BlockSpec((bn, H), lambda i: (i, 0))] * (1 + nm)
                  + [pl.BlockSpec((H, H), lambda i: (0, 0))] * (1 + nm)
                  + [pl.BlockSpec((1, H), lambda i: (0, 0))])
    stats = pl.pallas_call(
        body_a,
        out_shape=jax.ShapeDtypeStruct((8, H), f32),
        grid=(steps,),
        in_specs=data_specs,
        out_specs=pl.BlockSpec((8, H), lambda i: (0, 0)),
        scratch_shapes=[pltpu.VMEM((8, H), f32)],
    )(x, *mes_list, *wu_parts, bu)

    return pl.pallas_call(
        body_b,
        out_shape=jax.ShapeDtypeStruct((n, H), f32),
        grid=(steps,),
        in_specs=data_specs + [pl.BlockSpec((1, H), lambda i: (0, 0))] * 2
        + [pl.BlockSpec((8, H), lambda i: (0, 0))],
        out_specs=pl.BlockSpec((bn, H), lambda i: (i, 0)),
    )(x, *mes_list, *wu_parts, bu, gu, bbu, stats)


def _pad_idx(idx, e_pad):
    e = idx.shape[0]
    pad = jnp.arange(e_pad - e, dtype=jnp.int32) % 256
    return jnp.concatenate([idx, pad]).reshape(e_pad // CH, CH)


def _pad_inv(inv, e_pad):
    e = inv.shape[0]
    return jnp.concatenate(
        [inv, jnp.zeros((e_pad - e, inv.shape[1]), inv.dtype)])


def _row(v):
    return v.reshape(1, -1)


def kernel(x_0, x_1, adj_0_0, adj_0_1, adj_1_1, inv_0_0, inv_0_1, inv_1_1,
           Wm_00, bm_00, gm_00, bb_00, Wi_00, bi_00,
           Wm_01, bm_01, gm_01, bb_01, Wi_01, bi_01,
           Wm_11, bm_11, gm_11, bb_11, Wi_11, bi_11,
           Wu_0, bu_0, gu_0, bbu_0, Wu_1, bu_1, gu_1, bbu_1):
    n0, n1 = x_0.shape[0], x_1.shape[0]
    e = adj_0_0.shape[1]
    assert e % CH == 0
    n_chunks = -(-e // CH)
    e_pad = -(-n_chunks // NW) * NW * CH
    np0 = -(-n0 // 2048) * 2048
    np1 = -(-n1 // 2048) * 2048

    ps00, pr00, ps01 = _proj3(x_0, jnp.concatenate(
        [Wm_00[:H], Wm_00[H:2 * H], Wm_01[:H]], axis=1))
    pr01, ps11, pr11 = _proj3(x_1, jnp.concatenate(
        [Wm_01[H:2 * H], Wm_11[:H], Wm_11[H:2 * H]], axis=1))

    def pad_w(w):
        return jnp.concatenate([w, jnp.zeros((8 - w.shape[0], H), f32)])
    c00 = _cmat1(e_pad, _pad_inv(inv_0_0, e_pad), pad_w(Wm_00[2 * H:]), _row(bm_00))
    c01 = _cmat1(e_pad, _pad_inv(inv_0_1, e_pad), pad_w(Wm_01[2 * H:]), _row(bm_01))
    c11 = _cmat1(e_pad, _pad_inv(inv_1_1, e_pad), pad_w(Wm_11[2 * H:]), _row(bm_11))

    idx = {
        "00": (_pad_idx(adj_0_0[0], e_pad), _pad_idx(adj_0_0[1], e_pad)),
        "01": (_pad_idx(adj_0_1[0], e_pad), _pad_idx(adj_0_1[1], e_pad)),
        "11": (_pad_idx(adj_1_1[0], e_pad), _pad_idx(adj_1_1[1], e_pad)),
    }

    p1 = _build_pass1(e_pad, e)
    pre00, st00 = p1(ps00, pr00, c00, idx["00"][0], idx["00"][1])
    pre01, st01 = p1(ps01, pr01, c01, idx["01"][0], idx["01"][1])
    pre11, st11 = p1(ps11, pr11, c11, idx["11"][0], idx["11"][1])

    cf00 = _coef1(float(e), st00, _row(gm_00), _row(bb_00),
                  jnp.broadcast_to(bi_00, (1, H)))
    cf01 = _coef1(float(e), st01, _row(gm_01), _row(bb_01),
                  jnp.broadcast_to(bi_01, (1, H)))
    cf11 = _coef1(float(e), st11, _row(gm_11), _row(bb_11),
                  jnp.broadcast_to(bi_11, (1, H)))

    def pad_wi(wi):
        return jnp.concatenate([wi, jnp.zeros((H, H - 1), f32)], axis=1)
    val00 = _val(e_pad, e, pre00, cf00, pad_wi(Wi_00))
    val01 = _val(e_pad, e, pre01, cf01, pad_wi(Wi_01))
    val11 = _val(e_pad, e, pre11, cf11, pad_wi(Wi_11))

    p3_0 = _build_pass3(e_pad, np0)
    p3_1 = _build_pass3(e_pad, np1)
    mes00 = p3_0(val00, idx["00"][1])
    mes01 = p3_1(val01, idx["01"][1])
    mes11 = p3_1(val11, idx["11"][1])

    out0 = _update(x_0, [mes00], [Wu_0[:H], Wu_0[H:]],
                   _row(bu_0), _row(gu_0), _row(bbu_0))
    out1 = _update(x_1, [mes01, mes11], [Wu_1[:H], Wu_1[H:2 * H], Wu_1[2 * H:]],
                   _row(bu_1), _row(gu_1), _row(bbu_1))
    return (out0, out1)


# speedup vs baseline: 1.9607x; 1.0109x over previous
import functools

import jax
import jax.numpy as jnp
from jax import lax
from jax.experimental import pallas as pl
from jax.experimental.pallas import tpu as pltpu
from jax.experimental.pallas import tpu_sc as plsc

EPS = 1e-5
H = 128
NC, NS, LANES = 2, 16, 16
NW = NC * NS
CH = 80
SUP = 1280
f32 = jnp.float32


def _sc_mesh():
    return plsc.VectorSubcoreMesh(core_axis_name="c", subcore_axis_name="s",
                                  num_cores=NC, num_subcores=NS)


_SC_PARAMS = pltpu.CompilerParams(use_tc_tiling_on_sc=False)


def _build_pass1(e_pad, e_valid):
    ept = e_pad // NW
    n_chunks = ept // CH
    assert n_chunks % 2 == 0

    def body(ps_hbm, pr_hbm, c_hbm, i0_hbm, i1_hbm, pre_hbm, st_hbm,
             i0a, i1a, psa, pra, ca, i0b, i1b, psb, prb, cb, st_v,
             sa1, sa2, sa3, sb1, sb2, sb3):
        cid = lax.axis_index("c")
        sid = lax.axis_index("s")
        wid = sid * NC + cid
        chunk0 = wid * n_chunks
        for v in range(16):
            st_v[pl.ds(v * LANES, LANES)] = jnp.zeros((LANES,), f32)

        def issue(l, i0v, i1v, psv, prv, cv, s1, s2, s3):
            ci = chunk0 + l
            pltpu.sync_copy(i0_hbm.at[ci], i0v)
            pltpu.sync_copy(i1_hbm.at[ci], i1v)
            pltpu.async_copy(ps_hbm.at[i0v], psv, s1)
            pltpu.async_copy(pr_hbm.at[i1v], prv, s2)
            pltpu.async_copy(c_hbm.at[pl.ds(ci * CH, CH), :], cv, s3)

        def wait(l, i0v, i1v, psv, prv, cv, s1, s2, s3):
            ci = chunk0 + l
            pltpu.make_async_copy(ps_hbm.at[i0v], psv, s1).wait()
            pltpu.make_async_copy(pr_hbm.at[i1v], prv, s2).wait()
            pltpu.make_async_copy(
                c_hbm.at[pl.ds(ci * CH, CH), :], cv, s3).wait()

        def compute(l, psv, prv, cv):
            e0 = (chunk0 + l) * CH

            def row(j, carry):
                acc = list(carry)
                for v in range(8):
                    sl = pl.ds(v * LANES, LANES)
                    p = psv[j, sl] + prv[j, sl] + cv[j, sl]
                    psv[j, sl] = p
                    acc[v] = acc[v] + p
                    acc[8 + v] = acc[8 + v] + p * p
                return tuple(acc)

            zero16 = tuple(jnp.zeros((LANES,), f32) for _ in range(16))
            acc = lax.fori_loop(0, CH, row, zero16)

            @pl.when(e0 < e_valid)
            def _():
                for v in range(16):
                    sl = pl.ds(v * LANES, LANES)
                    st_v[sl] = st_v[sl] + acc[v]

            pltpu.sync_copy(psv, pre_hbm.at[pl.ds(e0, CH), :])

        bufa = (i0a, i1a, psa, pra, ca, sa1, sa2, sa3)
        bufb = (i0b, i1b, psb, prb, cb, sb1, sb2, sb3)
        last = n_chunks - 1

        issue(0, *bufa)

        def pair(it, _):
            la = 2 * it
            wait(la, *bufa)
            issue(la + 1, *bufb)
            compute(la, psa, pra, ca)
            wait(la + 1, *bufb)
            issue(jnp.minimum(la + 2, last), *bufa)
            compute(la + 1, psb, prb, cb)
            return 0

        lax.fori_loop(0, n_chunks // 2, pair, 0)
        wait(last, *bufa)
        pltpu.sync_copy(st_v, st_hbm.at[wid])

    return pl.kernel(
        body,
        out_type=[jax.ShapeDtypeStruct((e_pad, H), f32),
                  jax.ShapeDtypeStruct((NW, 2 * H), f32)],
        mesh=_sc_mesh(),
        scratch_types=[
            pltpu.VMEM((CH,), jnp.int32), pltpu.VMEM((CH,), jnp.int32),
            pltpu.VMEM((CH, H), f32), pltpu.VMEM((CH, H), f32),
            pltpu.VMEM((CH, H), f32),
            pltpu.VMEM((CH,), jnp.int32), pltpu.VMEM((CH,), jnp.int32),
            pltpu.VMEM((CH, H), f32), pltpu.VMEM((CH, H), f32),
            pltpu.VMEM((CH, H), f32),
            pltpu.VMEM((2 * H,), f32),
            pltpu.SemaphoreType.DMA, pltpu.SemaphoreType.DMA,
            pltpu.SemaphoreType.DMA, pltpu.SemaphoreType.DMA,
            pltpu.SemaphoreType.DMA, pltpu.SemaphoreType.DMA,
        ],
        compiler_params=_SC_PARAMS)


def _build_pass3(e_pad, n_pad):
    ept = e_pad // NS
    n_sup, tail = divmod(ept, SUP)
    rpt = n_pad // NS
    zr = rpt // 8

    def body(val_hbm, i1_hbm, mes_hbm, idx_sb, val_sb, zero_v, acc_sh, sem):
        cid = lax.axis_index("c")
        sid = lax.axis_index("s")
        base = sid * ept
        cbase = base // CH
        r0 = sid * rpt

        def zrow(j, _):
            zero_v[j] = jnp.zeros((LANES,), f32)
            return 0
        lax.fori_loop(0, zr, zrow, 0)
        for z in range(8):
            pltpu.sync_copy(zero_v, acc_sh.at[pl.ds(r0 + z * zr, zr)])
        plsc.subcore_barrier()

        for gl in range(4):
            g = cid * 4 + gl
            col = g * LANES

            def do_super(c0, e0, n):
                nch = n // CH
                pltpu.sync_copy(i1_hbm.at[pl.ds(c0, nch)],
                                idx_sb.at[pl.ds(0, nch)])
                cp = pltpu.async_copy(
                    val_hbm.at[pl.ds(e0, n), pl.ds(col, LANES)],
                    val_sb.at[pl.ds(0, n)], sem)
                cp.wait()
                for j in range(nch):
                    pltpu.sync_copy(val_sb.at[pl.ds(j * CH, CH)],
                                    acc_sh.at[idx_sb.at[j]], add=True)

            def sup_loop(s, _):
                do_super(cbase + s * (SUP // CH), base + s * SUP, SUP)
                return 0
            lax.fori_loop(0, n_sup, sup_loop, 0)
            if tail:
                do_super(cbase + n_sup * (SUP // CH), base + n_sup * SUP, tail)

            plsc.subcore_barrier()
            pltpu.sync_copy(acc_sh.at[pl.ds(r0, rpt)],
                            mes_hbm.at[pl.ds(r0, rpt), pl.ds(col, LANES)])
            for z in range(8):
                pltpu.sync_copy(zero_v, acc_sh.at[pl.ds(r0 + z * zr, zr)])
            plsc.subcore_barrier()

    return pl.kernel(
        body,
        out_type=jax.ShapeDtypeStruct((n_pad, H), f32),
        mesh=_sc_mesh(),
        scratch_types=[
            pltpu.VMEM((SUP // CH, CH), jnp.int32),
            pltpu.VMEM((SUP, LANES), f32),
            pltpu.VMEM((zr, LANES), f32),
            pltpu.VMEM_SHARED((n_pad, LANES), f32),
            pltpu.SemaphoreType.DMA,
        ],
        compiler_params=_SC_PARAMS)


def _proj3(x, wcat):
    n = x.shape[0]
    bn = 1000

    def body(x_ref, w_ref, o0, o1, o2):
        xb = x_ref[...]
        o0[...] = jnp.dot(xb, w_ref[:, 0:H], preferred_element_type=f32)
        o1[...] = jnp.dot(xb, w_ref[:, H:2 * H], preferred_element_type=f32)
        o2[...] = jnp.dot(xb, w_ref[:, 2 * H:3 * H], preferred_element_type=f32)

    return pl.pallas_call(
        body,
        out_shape=[jax.ShapeDtypeStruct((n, H), f32)] * 3,
        grid=(n // bn,),
        in_specs=[pl.BlockSpec((bn, H), lambda i: (i, 0)),
                  pl.BlockSpec((H, 3 * H), lambda i: (0, 0))],
        out_specs=[pl.BlockSpec((bn, H), lambda i: (i, 0))] * 3,
    )(x, wcat)


def _cmat1(e_pad, inv, wv, bm):
    be = 1280
    ninv = inv.shape[1]

    def body(inv_ref, wv_ref, bm_ref, o_ref):
        acc = jnp.broadcast_to(bm_ref[...], (be, H))
        for j in range(ninv):
            acc = acc + inv_ref[:, j:j + 1] * wv_ref[j:j + 1, :]
        o_ref[...] = acc

    return pl.pallas_call(
        body,
        out_shape=jax.ShapeDtypeStruct((e_pad, H), f32),
        grid=(e_pad // be,),
        in_specs=[pl.BlockSpec((be, ninv), lambda i: (i, 0)),
                  pl.BlockSpec((8, H), lambda i: (0, 0)),
                  pl.BlockSpec((1, H), lambda i: (0, 0))],
        out_specs=pl.BlockSpec((be, H), lambda i: (i, 0)),
    )(inv, wv, bm)


def _coef1(e_valid, st, gm, bb, bi):
    def body(st_ref, gm_ref, bb_ref, bi_ref, o_ref):
        s = st_ref[...]
        mean = jnp.sum(s[:, :H], axis=0, keepdims=True) / e_valid
        var = jnp.sum(s[:, H:], axis=0, keepdims=True) / e_valid - mean * mean
        scale = gm_ref[...] * lax.rsqrt(var + EPS)
        shift = bb_ref[...] - mean * scale
        o_ref[...] = jnp.concatenate(
            [scale, shift, bi_ref[...], jnp.zeros((5, H), f32)], axis=0)

    return pl.pallas_call(
        body,
        out_shape=jax.ShapeDtypeStruct((8, H), f32),
        in_specs=[pl.BlockSpec((NW, 2 * H), lambda: (0, 0))]
        + [pl.BlockSpec((1, H), lambda: (0, 0))] * 3,
        out_specs=pl.BlockSpec((8, H), lambda: (0, 0)),
    )(st, gm, bb, bi)


def _val(e_pad, e_valid, pre, coef, wi_pad):
    br = 512

    def body(pre_ref, coef_ref, wi_ref, o_ref):
        i = pl.program_id(0)
        scale = coef_ref[0:1, :]
        shift = coef_ref[1:2, :]
        t = pre_ref[...] * scale + shift
        msg = t * jax.nn.sigmoid(t)
        z = jnp.dot(msg, wi_ref[...], preferred_element_type=f32)
        w = jax.nn.sigmoid(z[:, 0:1] + coef_ref[2:3, 0:1])
        rows = i * br + lax.broadcasted_iota(jnp.int32, (br, 1), 0)
        w = jnp.where(rows < e_valid, w, 0.0)
        o_ref[...] = msg * w

    return pl.pallas_call(
        body,
        out_shape=jax.ShapeDtypeStruct((e_pad, H), f32),
        grid=(e_pad // br,),
        in_specs=[pl.BlockSpec((br, H), lambda i: (i, 0)),
                  pl.BlockSpec((8, H), lambda i: (0, 0)),
                  pl.BlockSpec((H, H), lambda i: (0, 0))],
        out_specs=pl.BlockSpec((br, H), lambda i: (i, 0)),
    )(pre, coef, wi_pad)


def _update(x, mes_list, wu_parts, bu, gu, bbu):
    n = x.shape[0]
    bn = 1000
    steps = n // bn
    nm = len(mes_list)

    def ubody(refs):
        x_ref = refs[0]
        m_refs = refs[1:1 + nm]
        w_refs = refs[1 + nm:2 + 2 * nm]
        bu_ref = refs[2 + 2 * nm]
        u = jnp.dot(x_ref[...], w_refs[0][...], preferred_element_type=f32)
        for k in range(nm):
            u = u + jnp.dot(m_refs[k][...], w_refs[1 + k][...],
                            preferred_element_type=f32)
        return u + bu_ref[...]

    def body_a(*refs):
        i = pl.program_id(0)
        o_ref, acc_ref = refs[-2], refs[-1]
        u = ubody(refs[:-2])

        @pl.when(i == 0)
        def _():
            acc_ref[...] = jnp.zeros((8, H), f32)

        acc_ref[0:1, :] += jnp.sum(u, axis=0, keepdims=True)
        acc_ref[1:2, :] += jnp.sum(u * u, axis=0, keepdims=True)

        @pl.when(i == steps - 1)
        def _():
            o_ref[...] = acc_ref[...]

    def body_b(*refs):
        gu_ref, bbu_ref, st_ref, o_ref = refs[-4:]
        u = ubody(refs[:-4])
        mean = st_ref[0:1, :] / n
        var = st_ref[1:2, :] / n - mean * mean
        scale = gu_ref[...] * lax.rsqrt(var + EPS)
        shift = bbu_ref[...] - mean * scale
        o_ref[...] = refs[0][...] + u * scale + shift

    data_specs = ([pl.BlockSpec((bn, H), lambda i: (i, 0))] * (1 + nm)
                  + [pl.BlockSpec((H, H), lambda i: (0, 0))] * (1 + nm)
                  + [pl.BlockSpec((1, H), lambda i: (0, 0))])
    stats = pl.pallas_call(
        body_a,
        out_shape=jax.ShapeDtypeStruct((8, H), f32),
        grid=(steps,),
        in_specs=data_specs,
        out_specs=pl.BlockSpec((8, H), lambda i: (0, 0)),
        scratch_shapes=[pltpu.VMEM((8, H), f32)],
    )(x, *mes_list, *wu_parts, bu)

    return pl.pallas_call(
        body_b,
        out_shape=jax.ShapeDtypeStruct((n, H), f32),
        grid=(steps,),
        in_specs=data_specs + [pl.BlockSpec((1, H), lambda i: (0, 0))] * 2
        + [pl.BlockSpec((8, H), lambda i: (0, 0))],
        out_specs=pl.BlockSpec((bn, H), lambda i: (i, 0)),
    )(x, *mes_list, *wu_parts, bu, gu, bbu, stats)


def _pad_idx(idx, e_pad):
    e = idx.shape[0]
    pad = jnp.arange(e_pad - e, dtype=jnp.int32) % 256
    return jnp.concatenate([idx, pad]).reshape(e_pad // CH, CH)


def _pad_inv(inv, e_pad):
    e = inv.shape[0]
    return jnp.concatenate(
        [inv, jnp.zeros((e_pad - e, inv.shape[1]), inv.dtype)])


def _row(v):
    return v.reshape(1, -1)


def kernel(x_0, x_1, adj_0_0, adj_0_1, adj_1_1, inv_0_0, inv_0_1, inv_1_1,
           Wm_00, bm_00, gm_00, bb_00, Wi_00, bi_00,
           Wm_01, bm_01, gm_01, bb_01, Wi_01, bi_01,
           Wm_11, bm_11, gm_11, bb_11, Wi_11, bi_11,
           Wu_0, bu_0, gu_0, bbu_0, Wu_1, bu_1, gu_1, bbu_1):
    n0, n1 = x_0.shape[0], x_1.shape[0]
    e = adj_0_0.shape[1]
    assert e % CH == 0
    grain = NW * CH
    cpw = -(-e // grain)
    cpw = cpw + (cpw % 2)
    e_pad = cpw * grain
    np0 = -(-n0 // 2048) * 2048
    np1 = -(-n1 // 2048) * 2048

    ps00, pr00, ps01 = _proj3(x_0, jnp.concatenate(
        [Wm_00[:H], Wm_00[H:2 * H], Wm_01[:H]], axis=1))
    pr01, ps11, pr11 = _proj3(x_1, jnp.concatenate(
        [Wm_01[H:2 * H], Wm_11[:H], Wm_11[H:2 * H]], axis=1))

    def pad_w(w):
        return jnp.concatenate([w, jnp.zeros((8 - w.shape[0], H), f32)])
    c00 = _cmat1(e_pad, _pad_inv(inv_0_0, e_pad), pad_w(Wm_00[2 * H:]), _row(bm_00))
    c01 = _cmat1(e_pad, _pad_inv(inv_0_1, e_pad), pad_w(Wm_01[2 * H:]), _row(bm_01))
    c11 = _cmat1(e_pad, _pad_inv(inv_1_1, e_pad), pad_w(Wm_11[2 * H:]), _row(bm_11))

    idx = {
        "00": (_pad_idx(adj_0_0[0], e_pad), _pad_idx(adj_0_0[1], e_pad)),
        "01": (_pad_idx(adj_0_1[0], e_pad), _pad_idx(adj_0_1[1], e_pad)),
        "11": (_pad_idx(adj_1_1[0], e_pad), _pad_idx(adj_1_1[1], e_pad)),
    }

    p1 = _build_pass1(e_pad, e)
    pre00, st00 = p1(ps00, pr00, c00, idx["00"][0], idx["00"][1])
    pre01, st01 = p1(ps01, pr01, c01, idx["01"][0], idx["01"][1])
    pre11, st11 = p1(ps11, pr11, c11, idx["11"][0], idx["11"][1])

    cf00 = _coef1(float(e), st00, _row(gm_00), _row(bb_00),
                  jnp.broadcast_to(bi_00, (1, H)))
    cf01 = _coef1(float(e), st01, _row(gm_01), _row(bb_01),
                  jnp.broadcast_to(bi_01, (1, H)))
    cf11 = _coef1(float(e), st11, _row(gm_11), _row(bb_11),
                  jnp.broadcast_to(bi_11, (1, H)))

    def pad_wi(wi):
        return jnp.concatenate([wi, jnp.zeros((H, H - 1), f32)], axis=1)
    val00 = _val(e_pad, e, pre00, cf00, pad_wi(Wi_00))
    val01 = _val(e_pad, e, pre01, cf01, pad_wi(Wi_01))
    val11 = _val(e_pad, e, pre11, cf11, pad_wi(Wi_11))

    p3_0 = _build_pass3(e_pad, np0)
    p3_1 = _build_pass3(e_pad, np1)
    mes00 = p3_0(val00, idx["00"][1])
    mes01 = p3_1(val01, idx["01"][1])
    mes11 = p3_1(val11, idx["11"][1])

    out0 = _update(x_0, [mes00], [Wu_0[:H], Wu_0[H:]],
                   _row(bu_0), _row(gu_0), _row(bbu_0))
    out1 = _update(x_1, [mes01, mes11], [Wu_1[:H], Wu_1[H:2 * H], Wu_1[2 * H:]],
                   _row(bu_1), _row(gu_1), _row(bbu_1))
    return (out0, out1)


# speedup vs baseline: 2.1787x; 1.1112x over previous
import functools

import jax
import jax.numpy as jnp
from jax import lax
from jax.experimental import pallas as pl
from jax.experimental.pallas import tpu as pltpu
from jax.experimental.pallas import tpu_sc as plsc

EPS = 1e-5
H = 128
NC, NS, LANES = 2, 16, 16
NW = NC * NS
CH = 80
SUP = 1280
f32 = jnp.float32


def _sc_mesh():
    return plsc.VectorSubcoreMesh(core_axis_name="c", subcore_axis_name="s",
                                  num_cores=NC, num_subcores=NS)


_SC_PARAMS = pltpu.CompilerParams(use_tc_tiling_on_sc=False)


def _build_pass1(e_pad, e_valid):
    ept = e_pad // NW
    n_chunks = ept // CH
    assert n_chunks % 2 == 0

    def body(ps_hbm, pr_hbm, c_hbm, i0_hbm, i1_hbm, pre_hbm, st_hbm,
             i0a, i1a, psa, pra, ca, i0b, i1b, psb, prb, cb, st_v,
             sa1, sa2, sa3, sb1, sb2, sb3):
        cid = lax.axis_index("c")
        sid = lax.axis_index("s")
        wid = sid * NC + cid
        chunk0 = wid * n_chunks
        for v in range(16):
            st_v[pl.ds(v * LANES, LANES)] = jnp.zeros((LANES,), f32)

        def issue(l, i0v, i1v, psv, prv, cv, s1, s2, s3):
            ci = chunk0 + l
            pltpu.sync_copy(i0_hbm.at[ci], i0v)
            pltpu.sync_copy(i1_hbm.at[ci], i1v)
            pltpu.async_copy(ps_hbm.at[i0v], psv, s1)
            pltpu.async_copy(pr_hbm.at[i1v], prv, s2)
            pltpu.async_copy(c_hbm.at[pl.ds(ci * CH, CH), :], cv, s3)

        def wait(l, i0v, i1v, psv, prv, cv, s1, s2, s3):
            ci = chunk0 + l
            pltpu.make_async_copy(ps_hbm.at[i0v], psv, s1).wait()
            pltpu.make_async_copy(pr_hbm.at[i1v], prv, s2).wait()
            pltpu.make_async_copy(
                c_hbm.at[pl.ds(ci * CH, CH), :], cv, s3).wait()

        def compute(l, psv, prv, cv):
            e0 = (chunk0 + l) * CH

            def row(j, carry):
                acc = list(carry)
                for v in range(8):
                    sl = pl.ds(v * LANES, LANES)
                    p = psv[j, sl] + prv[j, sl] + cv[j, sl]
                    psv[j, sl] = p
                    acc[v] = acc[v] + p
                    acc[8 + v] = acc[8 + v] + p * p
                return tuple(acc)

            zero16 = tuple(jnp.zeros((LANES,), f32) for _ in range(16))
            acc = lax.fori_loop(0, CH, row, zero16)

            @pl.when(e0 < e_valid)
            def _():
                for v in range(16):
                    sl = pl.ds(v * LANES, LANES)
                    st_v[sl] = st_v[sl] + acc[v]

            pltpu.sync_copy(psv, pre_hbm.at[pl.ds(e0, CH), :])

        bufa = (i0a, i1a, psa, pra, ca, sa1, sa2, sa3)
        bufb = (i0b, i1b, psb, prb, cb, sb1, sb2, sb3)
        last = n_chunks - 1

        issue(0, *bufa)

        def pair(it, _):
            la = 2 * it
            wait(la, *bufa)
            issue(la + 1, *bufb)
            compute(la, psa, pra, ca)
            wait(la + 1, *bufb)
            issue(jnp.minimum(la + 2, last), *bufa)
            compute(la + 1, psb, prb, cb)
            return 0

        lax.fori_loop(0, n_chunks // 2, pair, 0)
        wait(last, *bufa)
        pltpu.sync_copy(st_v, st_hbm.at[wid])

    return pl.kernel(
        body,
        out_type=[jax.ShapeDtypeStruct((e_pad, H), f32),
                  jax.ShapeDtypeStruct((NW, 2 * H), f32)],
        mesh=_sc_mesh(),
        scratch_types=[
            pltpu.VMEM((CH,), jnp.int32), pltpu.VMEM((CH,), jnp.int32),
            pltpu.VMEM((CH, H), f32), pltpu.VMEM((CH, H), f32),
            pltpu.VMEM((CH, H), f32),
            pltpu.VMEM((CH,), jnp.int32), pltpu.VMEM((CH,), jnp.int32),
            pltpu.VMEM((CH, H), f32), pltpu.VMEM((CH, H), f32),
            pltpu.VMEM((CH, H), f32),
            pltpu.VMEM((2 * H,), f32),
            pltpu.SemaphoreType.DMA, pltpu.SemaphoreType.DMA,
            pltpu.SemaphoreType.DMA, pltpu.SemaphoreType.DMA,
            pltpu.SemaphoreType.DMA, pltpu.SemaphoreType.DMA,
        ],
        compiler_params=_SC_PARAMS)


def _build_pass3(e_pad, n_pad):
    ept = e_pad // NS
    n_sup, tail = divmod(ept, SUP)
    rpt = n_pad // NS
    zr = rpt // 8

    def body(val_hbm, i1_hbm, mes_hbm, idx_sb, val_sb, zero_v, acc_sh, sem):
        cid = lax.axis_index("c")
        sid = lax.axis_index("s")
        base = sid * ept
        cbase = base // CH
        r0 = sid * rpt

        def zrow(j, _):
            zero_v[j] = jnp.zeros((LANES,), f32)
            return 0
        lax.fori_loop(0, zr, zrow, 0)
        for z in range(8):
            pltpu.sync_copy(zero_v, acc_sh.at[pl.ds(r0 + z * zr, zr)])
        plsc.subcore_barrier()

        for gl in range(4):
            g = cid * 4 + gl
            col = g * LANES

            def do_super(c0, e0, n):
                nch = n // CH
                pltpu.sync_copy(i1_hbm.at[pl.ds(c0, nch)],
                                idx_sb.at[pl.ds(0, nch)])
                cp = pltpu.async_copy(
                    val_hbm.at[pl.ds(e0, n), pl.ds(col, LANES)],
                    val_sb.at[pl.ds(0, n)], sem)
                cp.wait()
                for j in range(nch):
                    pltpu.sync_copy(val_sb.at[pl.ds(j * CH, CH)],
                                    acc_sh.at[idx_sb.at[j]], add=True)

            def sup_loop(s, _):
                do_super(cbase + s * (SUP // CH), base + s * SUP, SUP)
                return 0
            lax.fori_loop(0, n_sup, sup_loop, 0)
            if tail:
                do_super(cbase + n_sup * (SUP // CH), base + n_sup * SUP, tail)

            plsc.subcore_barrier()
            pltpu.sync_copy(acc_sh.at[pl.ds(r0, rpt)],
                            mes_hbm.at[pl.ds(r0, rpt), pl.ds(col, LANES)])
            for z in range(8):
                pltpu.sync_copy(zero_v, acc_sh.at[pl.ds(r0 + z * zr, zr)])
            plsc.subcore_barrier()

    return pl.kernel(
        body,
        out_type=jax.ShapeDtypeStruct((n_pad, H), f32),
        mesh=_sc_mesh(),
        scratch_types=[
            pltpu.VMEM((SUP // CH, CH), jnp.int32),
            pltpu.VMEM((SUP, LANES), f32),
            pltpu.VMEM((zr, LANES), f32),
            pltpu.VMEM_SHARED((n_pad, LANES), f32),
            pltpu.SemaphoreType.DMA,
        ],
        compiler_params=_SC_PARAMS)


def _proj3(x, wcat):
    n = x.shape[0]
    bn = 1000

    def body(x_ref, w_ref, o0, o1, o2):
        xb = x_ref[...]
        o0[...] = jnp.dot(xb, w_ref[:, 0:H], preferred_element_type=f32)
        o1[...] = jnp.dot(xb, w_ref[:, H:2 * H], preferred_element_type=f32)
        o2[...] = jnp.dot(xb, w_ref[:, 2 * H:3 * H], preferred_element_type=f32)

    return pl.pallas_call(
        body,
        out_shape=[jax.ShapeDtypeStruct((n, H), f32)] * 3,
        grid=(n // bn,),
        in_specs=[pl.BlockSpec((bn, H), lambda i: (i, 0)),
                  pl.BlockSpec((H, 3 * H), lambda i: (0, 0))],
        out_specs=[pl.BlockSpec((bn, H), lambda i: (i, 0))] * 3,
    )(x, wcat)


def _cmat1(e_pad, inv8, wv8, bm):
    be = 2560

    def body(inv_ref, wv_ref, bm_ref, o_ref):
        o_ref[...] = jnp.dot(inv_ref[...], wv_ref[...],
                             preferred_element_type=f32) + bm_ref[...]

    return pl.pallas_call(
        body,
        out_shape=jax.ShapeDtypeStruct((e_pad, H), f32),
        grid=(e_pad // be,),
        in_specs=[pl.BlockSpec((be, 8), lambda i: (i, 0)),
                  pl.BlockSpec((8, H), lambda i: (0, 0)),
                  pl.BlockSpec((1, H), lambda i: (0, 0))],
        out_specs=pl.BlockSpec((be, H), lambda i: (i, 0)),
    )(inv8, wv8, bm)


def _coef1(e_valid, st, gm, bb, bi):
    def body(st_ref, gm_ref, bb_ref, bi_ref, o_ref):
        s = st_ref[...]
        mean = jnp.sum(s[:, :H], axis=0, keepdims=True) / e_valid
        var = jnp.sum(s[:, H:], axis=0, keepdims=True) / e_valid - mean * mean
        scale = gm_ref[...] * lax.rsqrt(var + EPS)
        shift = bb_ref[...] - mean * scale
        o_ref[...] = jnp.concatenate(
            [scale, shift, bi_ref[...], jnp.zeros((5, H), f32)], axis=0)

    return pl.pallas_call(
        body,
        out_shape=jax.ShapeDtypeStruct((8, H), f32),
        in_specs=[pl.BlockSpec((NW, 2 * H), lambda: (0, 0))]
        + [pl.BlockSpec((1, H), lambda: (0, 0))] * 3,
        out_specs=pl.BlockSpec((8, H), lambda: (0, 0)),
    )(st, gm, bb, bi)


def _val(e_pad, e_valid, pre, coef, wi_pad):
    br = 512

    def body(pre_ref, coef_ref, wi_ref, o_ref):
        i = pl.program_id(0)
        scale = coef_ref[0:1, :]
        shift = coef_ref[1:2, :]
        t = pre_ref[...] * scale + shift
        msg = t * jax.nn.sigmoid(t)
        z = jnp.dot(msg, wi_ref[...], preferred_element_type=f32)
        w = jax.nn.sigmoid(z + coef_ref[2:3, :])
        rows = i * br + lax.broadcasted_iota(jnp.int32, (br, 1), 0)
        w = jnp.where(rows < e_valid, w, 0.0)
        o_ref[...] = msg * w

    return pl.pallas_call(
        body,
        out_shape=jax.ShapeDtypeStruct((e_pad, H), f32),
        grid=(e_pad // br,),
        in_specs=[pl.BlockSpec((br, H), lambda i: (i, 0)),
                  pl.BlockSpec((8, H), lambda i: (0, 0)),
                  pl.BlockSpec((H, H), lambda i: (0, 0))],
        out_specs=pl.BlockSpec((br, H), lambda i: (i, 0)),
    )(pre, coef, wi_pad)


def _update(x, mes_list, wu_parts, bu, gu, bbu):
    n = x.shape[0]
    bn = 1000
    steps = n // bn
    nm = len(mes_list)

    def ubody(refs):
        x_ref = refs[0]
        m_refs = refs[1:1 + nm]
        w_refs = refs[1 + nm:2 + 2 * nm]
        bu_ref = refs[2 + 2 * nm]
        u = jnp.dot(x_ref[...], w_refs[0][...], preferred_element_type=f32)
        for k in range(nm):
            u = u + jnp.dot(m_refs[k][...], w_refs[1 + k][...],
                            preferred_element_type=f32)
        return u + bu_ref[...]

    def body_a(*refs):
        i = pl.program_id(0)
        o_ref, acc_ref = refs[-2], refs[-1]
        u = ubody(refs[:-2])

        @pl.when(i == 0)
        def _():
            acc_ref[...] = jnp.zeros((8, H), f32)

        acc_ref[0:1, :] += jnp.sum(u, axis=0, keepdims=True)
        acc_ref[1:2, :] += jnp.sum(u * u, axis=0, keepdims=True)

        @pl.when(i == steps - 1)
        def _():
            o_ref[...] = acc_ref[...]

    def body_b(*refs):
        gu_ref, bbu_ref, st_ref, o_ref = refs[-4:]
        u = ubody(refs[:-4])
        mean = st_ref[0:1, :] / n
        var = st_ref[1:2, :] / n - mean * mean
        scale = gu_ref[...] * lax.rsqrt(var + EPS)
        shift = bbu_ref[...] - mean * scale
        o_ref[...] = refs[0][...] + u * scale + shift

    data_specs = ([pl.BlockSpec((bn, H), lambda i: (i, 0))] * (1 + nm)
                  + [pl.BlockSpec((H, H), lambda i: (0, 0))] * (1 + nm)
                  + [pl.BlockSpec((1, H), lambda i: (0, 0))])
    stats = pl.pallas_call(
        body_a,
        out_shape=jax.ShapeDtypeStruct((8, H), f32),
        grid=(steps,),
        in_specs=data_specs,
        out_specs=pl.BlockSpec((8, H), lambda i: (0, 0)),
        scratch_shapes=[pltpu.VMEM((8, H), f32)],
    )(x, *mes_list, *wu_parts, bu)

    return pl.pallas_call(
        body_b,
        out_shape=jax.ShapeDtypeStruct((n, H), f32),
        grid=(steps,),
        in_specs=data_specs + [pl.BlockSpec((1, H), lambda i: (0, 0))] * 2
        + [pl.BlockSpec((8, H), lambda i: (0, 0))],
        out_specs=pl.BlockSpec((bn, H), lambda i: (i, 0)),
    )(x, *mes_list, *wu_parts, bu, gu, bbu, stats)


def _pad_idx(idx, e_pad):
    e = idx.shape[0]
    pad = jnp.arange(e_pad - e, dtype=jnp.int32) % 256
    return jnp.concatenate([idx, pad]).reshape(e_pad // CH, CH)


def _pad_inv(inv, e_pad):
    e, ninv = inv.shape
    out = jnp.zeros((e_pad, 8), inv.dtype)
    return lax.dynamic_update_slice(out, inv, (0, 0))


def _row(v):
    return v.reshape(1, -1)


def kernel(x_0, x_1, adj_0_0, adj_0_1, adj_1_1, inv_0_0, inv_0_1, inv_1_1,
           Wm_00, bm_00, gm_00, bb_00, Wi_00, bi_00,
           Wm_01, bm_01, gm_01, bb_01, Wi_01, bi_01,
           Wm_11, bm_11, gm_11, bb_11, Wi_11, bi_11,
           Wu_0, bu_0, gu_0, bbu_0, Wu_1, bu_1, gu_1, bbu_1):
    n0, n1 = x_0.shape[0], x_1.shape[0]
    e = adj_0_0.shape[1]
    assert e % CH == 0
    grain = NW * CH
    cpw = -(-e // grain)
    cpw = cpw + (cpw % 2)
    e_pad = cpw * grain
    np0 = -(-n0 // 2048) * 2048
    np1 = -(-n1 // 2048) * 2048

    ps00, pr00, ps01 = _proj3(x_0, jnp.concatenate(
        [Wm_00[:H], Wm_00[H:2 * H], Wm_01[:H]], axis=1))
    pr01, ps11, pr11 = _proj3(x_1, jnp.concatenate(
        [Wm_01[H:2 * H], Wm_11[:H], Wm_11[H:2 * H]], axis=1))

    def pad_w(w):
        return jnp.concatenate([w, jnp.zeros((8 - w.shape[0], H), f32)])
    c00 = _cmat1(e_pad, _pad_inv(inv_0_0, e_pad), pad_w(Wm_00[2 * H:]), _row(bm_00))
    c01 = _cmat1(e_pad, _pad_inv(inv_0_1, e_pad), pad_w(Wm_01[2 * H:]), _row(bm_01))
    c11 = _cmat1(e_pad, _pad_inv(inv_1_1, e_pad), pad_w(Wm_11[2 * H:]), _row(bm_11))

    idx = {
        "00": (_pad_idx(adj_0_0[0], e_pad), _pad_idx(adj_0_0[1], e_pad)),
        "01": (_pad_idx(adj_0_1[0], e_pad), _pad_idx(adj_0_1[1], e_pad)),
        "11": (_pad_idx(adj_1_1[0], e_pad), _pad_idx(adj_1_1[1], e_pad)),
    }

    p1 = _build_pass1(e_pad, e)
    pre00, st00 = p1(ps00, pr00, c00, idx["00"][0], idx["00"][1])
    pre01, st01 = p1(ps01, pr01, c01, idx["01"][0], idx["01"][1])
    pre11, st11 = p1(ps11, pr11, c11, idx["11"][0], idx["11"][1])

    cf00 = _coef1(float(e), st00, _row(gm_00), _row(bb_00),
                  jnp.broadcast_to(bi_00, (1, H)))
    cf01 = _coef1(float(e), st01, _row(gm_01), _row(bb_01),
                  jnp.broadcast_to(bi_01, (1, H)))
    cf11 = _coef1(float(e), st11, _row(gm_11), _row(bb_11),
                  jnp.broadcast_to(bi_11, (1, H)))

    def pad_wi(wi):
        return jnp.tile(wi, (1, H))
    val00 = _val(e_pad, e, pre00, cf00, pad_wi(Wi_00))
    val01 = _val(e_pad, e, pre01, cf01, pad_wi(Wi_01))
    val11 = _val(e_pad, e, pre11, cf11, pad_wi(Wi_11))

    p3_0 = _build_pass3(e_pad, np0)
    p3_1 = _build_pass3(e_pad, np1)
    mes00 = p3_0(val00, idx["00"][1])
    mes01 = p3_1(val01, idx["01"][1])
    mes11 = p3_1(val11, idx["11"][1])

    out0 = _update(x_0, [mes00], [Wu_0[:H], Wu_0[H:]],
                   _row(bu_0), _row(gu_0), _row(bbu_0))
    out1 = _update(x_1, [mes01, mes11], [Wu_1[:H], Wu_1[H:2 * H], Wu_1[2 * H:]],
                   _row(bu_1), _row(gu_1), _row(bbu_1))
    return (out0, out1)


# speedup vs baseline: 2.2614x; 1.0380x over previous
import functools

import jax
import jax.numpy as jnp
from jax import lax
from jax.experimental import pallas as pl
from jax.experimental.pallas import tpu as pltpu
from jax.experimental.pallas import tpu_sc as plsc

EPS = 1e-5
H = 128
NC, NS, LANES = 2, 16, 16
NW = NC * NS
CH = 160
SUP = 1280
f32 = jnp.float32


def _sc_mesh():
    return plsc.VectorSubcoreMesh(core_axis_name="c", subcore_axis_name="s",
                                  num_cores=NC, num_subcores=NS)


_SC_PARAMS = pltpu.CompilerParams(use_tc_tiling_on_sc=False)


def _build_pass1(e_pad, e_valid):
    ept = e_pad // NW
    n_chunks = ept // CH
    assert n_chunks % 2 == 0

    def body(ps_hbm, pr_hbm, c_hbm, i0_hbm, i1_hbm, pre_hbm, st_hbm,
             i0a, i1a, psa, pra, ca, i0b, i1b, psb, prb, cb, st_v,
             sa1, sa2, sa3, sb1, sb2, sb3):
        cid = lax.axis_index("c")
        sid = lax.axis_index("s")
        wid = sid * NC + cid
        chunk0 = wid * n_chunks
        for v in range(16):
            st_v[pl.ds(v * LANES, LANES)] = jnp.zeros((LANES,), f32)

        def issue(l, i0v, i1v, psv, prv, cv, s1, s2, s3):
            ci = chunk0 + l
            pltpu.sync_copy(i0_hbm.at[ci], i0v)
            pltpu.sync_copy(i1_hbm.at[ci], i1v)
            pltpu.async_copy(ps_hbm.at[i0v], psv, s1)
            pltpu.async_copy(pr_hbm.at[i1v], prv, s2)
            pltpu.async_copy(c_hbm.at[pl.ds(ci * CH, CH), :], cv, s3)

        def wait(l, i0v, i1v, psv, prv, cv, s1, s2, s3):
            ci = chunk0 + l
            pltpu.make_async_copy(ps_hbm.at[i0v], psv, s1).wait()
            pltpu.make_async_copy(pr_hbm.at[i1v], prv, s2).wait()
            pltpu.make_async_copy(
                c_hbm.at[pl.ds(ci * CH, CH), :], cv, s3).wait()

        def compute(l, psv, prv, cv):
            e0 = (chunk0 + l) * CH

            def row(j, carry):
                acc = list(carry)
                for v in range(8):
                    sl = pl.ds(v * LANES, LANES)
                    p = psv[j, sl] + prv[j, sl] + cv[j, sl]
                    psv[j, sl] = p
                    acc[v] = acc[v] + p
                    acc[8 + v] = acc[8 + v] + p * p
                return tuple(acc)

            zero16 = tuple(jnp.zeros((LANES,), f32) for _ in range(16))
            acc = lax.fori_loop(0, CH, row, zero16)

            @pl.when(e0 < e_valid)
            def _():
                for v in range(16):
                    sl = pl.ds(v * LANES, LANES)
                    st_v[sl] = st_v[sl] + acc[v]

            pltpu.sync_copy(psv, pre_hbm.at[pl.ds(e0, CH), :])

        bufa = (i0a, i1a, psa, pra, ca, sa1, sa2, sa3)
        bufb = (i0b, i1b, psb, prb, cb, sb1, sb2, sb3)
        last = n_chunks - 1

        issue(0, *bufa)

        def pair(it, _):
            la = 2 * it
            wait(la, *bufa)
            issue(la + 1, *bufb)
            compute(la, psa, pra, ca)
            wait(la + 1, *bufb)
            issue(jnp.minimum(la + 2, last), *bufa)
            compute(la + 1, psb, prb, cb)
            return 0

        lax.fori_loop(0, n_chunks // 2, pair, 0)
        wait(last, *bufa)
        pltpu.sync_copy(st_v, st_hbm.at[wid])

    return pl.kernel(
        body,
        out_type=[jax.ShapeDtypeStruct((e_pad, H), f32),
                  jax.ShapeDtypeStruct((NW, 2 * H), f32)],
        mesh=_sc_mesh(),
        scratch_types=[
            pltpu.VMEM((CH,), jnp.int32), pltpu.VMEM((CH,), jnp.int32),
            pltpu.VMEM((CH, H), f32), pltpu.VMEM((CH, H), f32),
            pltpu.VMEM((CH, H), f32),
            pltpu.VMEM((CH,), jnp.int32), pltpu.VMEM((CH,), jnp.int32),
            pltpu.VMEM((CH, H), f32), pltpu.VMEM((CH, H), f32),
            pltpu.VMEM((CH, H), f32),
            pltpu.VMEM((2 * H,), f32),
            pltpu.SemaphoreType.DMA, pltpu.SemaphoreType.DMA,
            pltpu.SemaphoreType.DMA, pltpu.SemaphoreType.DMA,
            pltpu.SemaphoreType.DMA, pltpu.SemaphoreType.DMA,
        ],
        compiler_params=_SC_PARAMS)


def _build_pass3(e_pad, n_pad):
    ept = e_pad // NS
    n_sup, tail = divmod(ept, SUP)
    assert tail == 0 and n_sup % 2 == 0
    spc = SUP // CH
    rpt = n_pad // NS
    zr = rpt // 8

    def body(val_hbm, i1_hbm, mes_hbm, idxa, vala, idxb, valb, zero_v,
             acc_sh, sema, semb):
        cid = lax.axis_index("c")
        sid = lax.axis_index("s")
        base = sid * ept
        cbase = base // CH
        r0 = sid * rpt

        def zrow(j, _):
            zero_v[j] = jnp.zeros((LANES,), f32)
            return 0
        lax.fori_loop(0, zr, zrow, 0)
        for z in range(8):
            pltpu.sync_copy(zero_v, acc_sh.at[pl.ds(r0 + z * zr, zr)])
        plsc.subcore_barrier()

        last = n_sup - 1
        for gl in range(4):
            g = cid * 4 + gl
            col = g * LANES

            def load(s, idxv, valv, sem):
                pltpu.sync_copy(i1_hbm.at[pl.ds(cbase + s * spc, spc)], idxv)
                pltpu.async_copy(
                    val_hbm.at[pl.ds(base + s * SUP, SUP), pl.ds(col, LANES)],
                    valv, sem)

            def wait_load(s, valv, sem):
                pltpu.make_async_copy(
                    val_hbm.at[pl.ds(base + s * SUP, SUP), pl.ds(col, LANES)],
                    valv, sem).wait()

            def scat(valv, idxv):
                for j in range(spc):
                    pltpu.sync_copy(valv.at[pl.ds(j * CH, CH)],
                                    acc_sh.at[idxv.at[j]], add=True)

            load(0, idxa, vala, sema)

            def pair(it, _):
                sa = 2 * it
                wait_load(sa, vala, sema)
                load(sa + 1, idxb, valb, semb)
                scat(vala, idxa)
                wait_load(sa + 1, valb, semb)
                load(jnp.minimum(sa + 2, last), idxa, vala, sema)
                scat(valb, idxb)
                return 0

            lax.fori_loop(0, n_sup // 2, pair, 0)
            wait_load(last, vala, sema)

            plsc.subcore_barrier()
            pltpu.sync_copy(acc_sh.at[pl.ds(r0, rpt)],
                            mes_hbm.at[pl.ds(r0, rpt), pl.ds(col, LANES)])
            for z in range(8):
                pltpu.sync_copy(zero_v, acc_sh.at[pl.ds(r0 + z * zr, zr)])
            plsc.subcore_barrier()

    return pl.kernel(
        body,
        out_type=jax.ShapeDtypeStruct((n_pad, H), f32),
        mesh=_sc_mesh(),
        scratch_types=[
            pltpu.VMEM((SUP // CH, CH), jnp.int32),
            pltpu.VMEM((SUP, LANES), f32),
            pltpu.VMEM((SUP // CH, CH), jnp.int32),
            pltpu.VMEM((SUP, LANES), f32),
            pltpu.VMEM((zr, LANES), f32),
            pltpu.VMEM_SHARED((n_pad, LANES), f32),
            pltpu.SemaphoreType.DMA, pltpu.SemaphoreType.DMA,
        ],
        compiler_params=_SC_PARAMS)


def _proj3(x, wcat):
    n = x.shape[0]
    bn = 1000

    def body(x_ref, w_ref, o0, o1, o2):
        xb = x_ref[...]
        o0[...] = jnp.dot(xb, w_ref[:, 0:H], preferred_element_type=f32)
        o1[...] = jnp.dot(xb, w_ref[:, H:2 * H], preferred_element_type=f32)
        o2[...] = jnp.dot(xb, w_ref[:, 2 * H:3 * H], preferred_element_type=f32)

    return pl.pallas_call(
        body,
        out_shape=[jax.ShapeDtypeStruct((n, H), f32)] * 3,
        grid=(n // bn,),
        in_specs=[pl.BlockSpec((bn, H), lambda i: (i, 0)),
                  pl.BlockSpec((H, 3 * H), lambda i: (0, 0))],
        out_specs=[pl.BlockSpec((bn, H), lambda i: (i, 0))] * 3,
    )(x, wcat)


def _cmat1(e_pad, inv8, wv8, bm):
    be = 2560

    def body(inv_ref, wv_ref, bm_ref, o_ref):
        o_ref[...] = jnp.dot(inv_ref[...], wv_ref[...],
                             preferred_element_type=f32) + bm_ref[...]

    return pl.pallas_call(
        body,
        out_shape=jax.ShapeDtypeStruct((e_pad, H), f32),
        grid=(e_pad // be,),
        in_specs=[pl.BlockSpec((be, 8), lambda i: (i, 0)),
                  pl.BlockSpec((8, H), lambda i: (0, 0)),
                  pl.BlockSpec((1, H), lambda i: (0, 0))],
        out_specs=pl.BlockSpec((be, H), lambda i: (i, 0)),
    )(inv8, wv8, bm)


def _coef1(e_valid, st, gm, bb, bi):
    def body(st_ref, gm_ref, bb_ref, bi_ref, o_ref):
        s = st_ref[...]
        mean = jnp.sum(s[:, :H], axis=0, keepdims=True) / e_valid
        var = jnp.sum(s[:, H:], axis=0, keepdims=True) / e_valid - mean * mean
        scale = gm_ref[...] * lax.rsqrt(var + EPS)
        shift = bb_ref[...] - mean * scale
        o_ref[...] = jnp.concatenate(
            [scale, shift, bi_ref[...], jnp.zeros((5, H), f32)], axis=0)

    return pl.pallas_call(
        body,
        out_shape=jax.ShapeDtypeStruct((8, H), f32),
        in_specs=[pl.BlockSpec((NW, 2 * H), lambda: (0, 0))]
        + [pl.BlockSpec((1, H), lambda: (0, 0))] * 3,
        out_specs=pl.BlockSpec((8, H), lambda: (0, 0)),
    )(st, gm, bb, bi)


def _val(e_pad, e_valid, pre, coef, wi_pad):
    br = 512

    def body(pre_ref, coef_ref, wi_ref, o_ref):
        i = pl.program_id(0)
        scale = coef_ref[0:1, :]
        shift = coef_ref[1:2, :]
        t = pre_ref[...] * scale + shift
        msg = t * jax.nn.sigmoid(t)
        z = jnp.dot(msg, wi_ref[...], preferred_element_type=f32)
        w = jax.nn.sigmoid(z + coef_ref[2:3, :])
        rows = i * br + lax.broadcasted_iota(jnp.int32, (br, 1), 0)
        w = jnp.where(rows < e_valid, w, 0.0)
        o_ref[...] = msg * w

    return pl.pallas_call(
        body,
        out_shape=jax.ShapeDtypeStruct((e_pad, H), f32),
        grid=(e_pad // br,),
        in_specs=[pl.BlockSpec((br, H), lambda i: (i, 0)),
                  pl.BlockSpec((8, H), lambda i: (0, 0)),
                  pl.BlockSpec((H, H), lambda i: (0, 0))],
        out_specs=pl.BlockSpec((br, H), lambda i: (i, 0)),
    )(pre, coef, wi_pad)


def _update(x, mes_list, wu_parts, bu, gu, bbu):
    n = x.shape[0]
    bn = 1000
    steps = n // bn
    nm = len(mes_list)

    def ubody(refs):
        x_ref = refs[0]
        m_refs = refs[1:1 + nm]
        w_refs = refs[1 + nm:2 + 2 * nm]
        bu_ref = refs[2 + 2 * nm]
        u = jnp.dot(x_ref[...], w_refs[0][...], preferred_element_type=f32)
        for k in range(nm):
            u = u + jnp.dot(m_refs[k][...], w_refs[1 + k][...],
                            preferred_element_type=f32)
        return u + bu_ref[...]

    def body_a(*refs):
        i = pl.program_id(0)
        o_ref, acc_ref = refs[-2], refs[-1]
        u = ubody(refs[:-2])

        @pl.when(i == 0)
        def _():
            acc_ref[...] = jnp.zeros((8, H), f32)

        acc_ref[0:1, :] += jnp.sum(u, axis=0, keepdims=True)
        acc_ref[1:2, :] += jnp.sum(u * u, axis=0, keepdims=True)

        @pl.when(i == steps - 1)
        def _():
            o_ref[...] = acc_ref[...]

    def body_b(*refs):
        gu_ref, bbu_ref, st_ref, o_ref = refs[-4:]
        u = ubody(refs[:-4])
        mean = st_ref[0:1, :] / n
        var = st_ref[1:2, :] / n - mean * mean
        scale = gu_ref[...] * lax.rsqrt(var + EPS)
        shift = bbu_ref[...] - mean * scale
        o_ref[...] = refs[0][...] + u * scale + shift

    data_specs = ([pl.BlockSpec((bn, H), lambda i: (i, 0))] * (1 + nm)
                  + [pl.BlockSpec((H, H), lambda i: (0, 0))] * (1 + nm)
                  + [pl.BlockSpec((1, H), lambda i: (0, 0))])
    stats = pl.pallas_call(
        body_a,
        out_shape=jax.ShapeDtypeStruct((8, H), f32),
        grid=(steps,),
        in_specs=data_specs,
        out_specs=pl.BlockSpec((8, H), lambda i: (0, 0)),
        scratch_shapes=[pltpu.VMEM((8, H), f32)],
    )(x, *mes_list, *wu_parts, bu)

    return pl.pallas_call(
        body_b,
        out_shape=jax.ShapeDtypeStruct((n, H), f32),
        grid=(steps,),
        in_specs=data_specs + [pl.BlockSpec((1, H), lambda i: (0, 0))] * 2
        + [pl.BlockSpec((8, H), lambda i: (0, 0))],
        out_specs=pl.BlockSpec((bn, H), lambda i: (i, 0)),
    )(x, *mes_list, *wu_parts, bu, gu, bbu, stats)


def _pad_idx(idx, e_pad):
    e = idx.shape[0]
    pad = jnp.arange(e_pad - e, dtype=jnp.int32) % 256
    return jnp.concatenate([idx, pad]).reshape(e_pad // CH, CH)


def _pad_inv(inv, e_pad):
    e, ninv = inv.shape
    out = jnp.zeros((e_pad, 8), inv.dtype)
    return lax.dynamic_update_slice(out, inv, (0, 0))


def _row(v):
    return v.reshape(1, -1)


def kernel(x_0, x_1, adj_0_0, adj_0_1, adj_1_1, inv_0_0, inv_0_1, inv_1_1,
           Wm_00, bm_00, gm_00, bb_00, Wi_00, bi_00,
           Wm_01, bm_01, gm_01, bb_01, Wi_01, bi_01,
           Wm_11, bm_11, gm_11, bb_11, Wi_11, bi_11,
           Wu_0, bu_0, gu_0, bbu_0, Wu_1, bu_1, gu_1, bbu_1):
    n0, n1 = x_0.shape[0], x_1.shape[0]
    e = adj_0_0.shape[1]
    assert e % CH == 0
    grain = NW * CH
    cpw = -(-e // grain)
    cpw = cpw + (cpw % 2)
    e_pad = cpw * grain
    np0 = -(-n0 // 2048) * 2048
    np1 = -(-n1 // 2048) * 2048

    ps00, pr00, ps01 = _proj3(x_0, jnp.concatenate(
        [Wm_00[:H], Wm_00[H:2 * H], Wm_01[:H]], axis=1))
    pr01, ps11, pr11 = _proj3(x_1, jnp.concatenate(
        [Wm_01[H:2 * H], Wm_11[:H], Wm_11[H:2 * H]], axis=1))

    def pad_w(w):
        return jnp.concatenate([w, jnp.zeros((8 - w.shape[0], H), f32)])
    c00 = _cmat1(e_pad, _pad_inv(inv_0_0, e_pad), pad_w(Wm_00[2 * H:]), _row(bm_00))
    c01 = _cmat1(e_pad, _pad_inv(inv_0_1, e_pad), pad_w(Wm_01[2 * H:]), _row(bm_01))
    c11 = _cmat1(e_pad, _pad_inv(inv_1_1, e_pad), pad_w(Wm_11[2 * H:]), _row(bm_11))

    idx = {
        "00": (_pad_idx(adj_0_0[0], e_pad), _pad_idx(adj_0_0[1], e_pad)),
        "01": (_pad_idx(adj_0_1[0], e_pad), _pad_idx(adj_0_1[1], e_pad)),
        "11": (_pad_idx(adj_1_1[0], e_pad), _pad_idx(adj_1_1[1], e_pad)),
    }

    p1 = _build_pass1(e_pad, e)
    pre00, st00 = p1(ps00, pr00, c00, idx["00"][0], idx["00"][1])
    pre01, st01 = p1(ps01, pr01, c01, idx["01"][0], idx["01"][1])
    pre11, st11 = p1(ps11, pr11, c11, idx["11"][0], idx["11"][1])

    cf00 = _coef1(float(e), st00, _row(gm_00), _row(bb_00),
                  jnp.broadcast_to(bi_00, (1, H)))
    cf01 = _coef1(float(e), st01, _row(gm_01), _row(bb_01),
                  jnp.broadcast_to(bi_01, (1, H)))
    cf11 = _coef1(float(e), st11, _row(gm_11), _row(bb_11),
                  jnp.broadcast_to(bi_11, (1, H)))

    def pad_wi(wi):
        return jnp.tile(wi, (1, H))
    val00 = _val(e_pad, e, pre00, cf00, pad_wi(Wi_00))
    val01 = _val(e_pad, e, pre01, cf01, pad_wi(Wi_01))
    val11 = _val(e_pad, e, pre11, cf11, pad_wi(Wi_11))

    p3_0 = _build_pass3(e_pad, np0)
    p3_1 = _build_pass3(e_pad, np1)
    mes00 = p3_0(val00, idx["00"][1])
    mes01 = p3_1(val01, idx["01"][1])
    mes11 = p3_1(val11, idx["11"][1])

    out0 = _update(x_0, [mes00], [Wu_0[:H], Wu_0[H:]],
                   _row(bu_0), _row(gu_0), _row(bbu_0))
    out1 = _update(x_1, [mes01, mes11], [Wu_1[:H], Wu_1[H:2 * H], Wu_1[2 * H:]],
                   _row(bu_1), _row(gu_1), _row(bbu_1))
    return (out0, out1)


# speedup vs baseline: 2.4961x; 1.1038x over previous
import functools

import jax
import jax.numpy as jnp
from jax import lax
from jax.experimental import pallas as pl
from jax.experimental.pallas import tpu as pltpu
from jax.experimental.pallas import tpu_sc as plsc

EPS = 1e-5
H = 128
NC, NS, LANES = 2, 16, 16
NW = NC * NS
CH = 160
SUP = 1280
f32 = jnp.float32


def _sc_mesh():
    return plsc.VectorSubcoreMesh(core_axis_name="c", subcore_axis_name="s",
                                  num_cores=NC, num_subcores=NS)


_SC_PARAMS = pltpu.CompilerParams(use_tc_tiling_on_sc=False)


def _build_pass1(e_pad, e_valid):
    ept = e_pad // NW
    n_chunks = ept // CH
    assert n_chunks % 2 == 0

    def body(ps_hbm, pr_hbm, c_hbm, i0_hbm, i1_hbm, pre_hbm, st_hbm,
             i0a, i1a, psa, pra, ca, i0b, i1b, psb, prb, cb, st_v,
             sa1, sa2, sa3, sb1, sb2, sb3):
        cid = lax.axis_index("c")
        sid = lax.axis_index("s")
        wid = sid * NC + cid
        chunk0 = wid * n_chunks
        for v in range(16):
            st_v[pl.ds(v * LANES, LANES)] = jnp.zeros((LANES,), f32)

        def issue(l, i0v, i1v, psv, prv, cv, s1, s2, s3):
            ci = chunk0 + l
            pltpu.sync_copy(i0_hbm.at[ci], i0v)
            pltpu.sync_copy(i1_hbm.at[ci], i1v)
            pltpu.async_copy(ps_hbm.at[i0v], psv, s1)
            pltpu.async_copy(pr_hbm.at[i1v], prv, s2)
            pltpu.async_copy(c_hbm.at[pl.ds(ci * CH, CH), :], cv, s3)

        def wait(l, i0v, i1v, psv, prv, cv, s1, s2, s3):
            ci = chunk0 + l
            pltpu.make_async_copy(ps_hbm.at[i0v], psv, s1).wait()
            pltpu.make_async_copy(pr_hbm.at[i1v], prv, s2).wait()
            pltpu.make_async_copy(
                c_hbm.at[pl.ds(ci * CH, CH), :], cv, s3).wait()

        def compute(l, psv, prv, cv):
            e0 = (chunk0 + l) * CH

            def row(j, carry):
                acc = list(carry)
                for v in range(8):
                    sl = pl.ds(v * LANES, LANES)
                    p = psv[j, sl] + prv[j, sl] + cv[j, sl]
                    psv[j, sl] = p
                    acc[v] = acc[v] + p
                    acc[8 + v] = acc[8 + v] + p * p
                return tuple(acc)

            zero16 = tuple(jnp.zeros((LANES,), f32) for _ in range(16))
            acc = lax.fori_loop(0, CH, row, zero16)

            @pl.when(e0 < e_valid)
            def _():
                for v in range(16):
                    sl = pl.ds(v * LANES, LANES)
                    st_v[sl] = st_v[sl] + acc[v]

            pltpu.sync_copy(psv, pre_hbm.at[pl.ds(e0, CH), :])

        bufa = (i0a, i1a, psa, pra, ca, sa1, sa2, sa3)
        bufb = (i0b, i1b, psb, prb, cb, sb1, sb2, sb3)
        last = n_chunks - 1

        issue(0, *bufa)

        def pair(it, _):
            la = 2 * it
            wait(la, *bufa)
            issue(la + 1, *bufb)
            compute(la, psa, pra, ca)
            wait(la + 1, *bufb)
            issue(jnp.minimum(la + 2, last), *bufa)
            compute(la + 1, psb, prb, cb)
            return 0

        lax.fori_loop(0, n_chunks // 2, pair, 0)
        wait(last, *bufa)
        pltpu.sync_copy(st_v, st_hbm.at[wid])

    return pl.kernel(
        body,
        out_type=[jax.ShapeDtypeStruct((e_pad, H), f32),
                  jax.ShapeDtypeStruct((NW, 2 * H), f32)],
        mesh=_sc_mesh(),
        scratch_types=[
            pltpu.VMEM((CH,), jnp.int32), pltpu.VMEM((CH,), jnp.int32),
            pltpu.VMEM((CH, H), f32), pltpu.VMEM((CH, H), f32),
            pltpu.VMEM((CH, H), f32),
            pltpu.VMEM((CH,), jnp.int32), pltpu.VMEM((CH,), jnp.int32),
            pltpu.VMEM((CH, H), f32), pltpu.VMEM((CH, H), f32),
            pltpu.VMEM((CH, H), f32),
            pltpu.VMEM((2 * H,), f32),
            pltpu.SemaphoreType.DMA, pltpu.SemaphoreType.DMA,
            pltpu.SemaphoreType.DMA, pltpu.SemaphoreType.DMA,
            pltpu.SemaphoreType.DMA, pltpu.SemaphoreType.DMA,
        ],
        compiler_params=_SC_PARAMS)


def _build_pass3(e_pad, n_pad):
    ept = e_pad // NS
    n_sup, tail = divmod(ept, SUP)
    assert tail == 0 and n_sup % 2 == 0
    spc = SUP // CH
    rpt = n_pad // NS
    zr = rpt // 8

    def body(val_hbm, i1_hbm, mes_hbm, idxa, vala, idxb, valb, zero_v,
             acc_sh, sema, semb):
        cid = lax.axis_index("c")
        sid = lax.axis_index("s")
        base = sid * ept
        cbase = base // CH
        r0 = sid * rpt

        def zrow(j, _):
            zero_v[j] = jnp.zeros((LANES,), f32)
            return 0
        lax.fori_loop(0, zr, zrow, 0)
        for z in range(8):
            pltpu.sync_copy(zero_v, acc_sh.at[pl.ds(r0 + z * zr, zr)])
        plsc.subcore_barrier()

        last = n_sup - 1
        for gl in range(4):
            g = cid * 4 + gl
            col = g * LANES

            def load(s, idxv, valv, sem):
                pltpu.sync_copy(i1_hbm.at[pl.ds(cbase + s * spc, spc)], idxv)
                pltpu.async_copy(
                    val_hbm.at[pl.ds(base + s * SUP, SUP), pl.ds(col, LANES)],
                    valv, sem)

            def wait_load(s, valv, sem):
                pltpu.make_async_copy(
                    val_hbm.at[pl.ds(base + s * SUP, SUP), pl.ds(col, LANES)],
                    valv, sem).wait()

            def scat(valv, idxv):
                for j in range(spc):
                    pltpu.sync_copy(valv.at[pl.ds(j * CH, CH)],
                                    acc_sh.at[idxv.at[j]], add=True)

            load(0, idxa, vala, sema)

            def pair(it, _):
                sa = 2 * it
                wait_load(sa, vala, sema)
                load(sa + 1, idxb, valb, semb)
                scat(vala, idxa)
                wait_load(sa + 1, valb, semb)
                load(jnp.minimum(sa + 2, last), idxa, vala, sema)
                scat(valb, idxb)
                return 0

            lax.fori_loop(0, n_sup // 2, pair, 0)
            wait_load(last, vala, sema)

            plsc.subcore_barrier()
            pltpu.sync_copy(acc_sh.at[pl.ds(r0, rpt)],
                            mes_hbm.at[pl.ds(r0, rpt), pl.ds(col, LANES)])
            for z in range(8):
                pltpu.sync_copy(zero_v, acc_sh.at[pl.ds(r0 + z * zr, zr)])
            plsc.subcore_barrier()

    return pl.kernel(
        body,
        out_type=jax.ShapeDtypeStruct((n_pad, H), f32),
        mesh=_sc_mesh(),
        scratch_types=[
            pltpu.VMEM((SUP // CH, CH), jnp.int32),
            pltpu.VMEM((SUP, LANES), f32),
            pltpu.VMEM((SUP // CH, CH), jnp.int32),
            pltpu.VMEM((SUP, LANES), f32),
            pltpu.VMEM((zr, LANES), f32),
            pltpu.VMEM_SHARED((n_pad, LANES), f32),
            pltpu.SemaphoreType.DMA, pltpu.SemaphoreType.DMA,
        ],
        compiler_params=_SC_PARAMS)


def _proj3(x, wcat):
    n = x.shape[0]
    bn = 1000

    def body(x_ref, w_ref, o0, o1, o2):
        xb = x_ref[...]
        o0[...] = jnp.dot(xb, w_ref[:, 0:H], preferred_element_type=f32)
        o1[...] = jnp.dot(xb, w_ref[:, H:2 * H], preferred_element_type=f32)
        o2[...] = jnp.dot(xb, w_ref[:, 2 * H:3 * H], preferred_element_type=f32)

    return pl.pallas_call(
        body,
        out_shape=[jax.ShapeDtypeStruct((n, H), f32)] * 3,
        grid=(n // bn,),
        in_specs=[pl.BlockSpec((bn, H), lambda i: (i, 0)),
                  pl.BlockSpec((H, 3 * H), lambda i: (0, 0))],
        out_specs=[pl.BlockSpec((bn, H), lambda i: (i, 0))] * 3,
    )(x, wcat)


def _cmat1(e_pad, inv8, wv8, bm):
    be = 2560

    def body(inv_ref, wv_ref, bm_ref, o_ref):
        o_ref[...] = jnp.dot(inv_ref[...], wv_ref[...],
                             preferred_element_type=f32) + bm_ref[...]

    return pl.pallas_call(
        body,
        out_shape=jax.ShapeDtypeStruct((e_pad, H), f32),
        grid=(e_pad // be,),
        in_specs=[pl.BlockSpec((be, 8), lambda i: (i, 0)),
                  pl.BlockSpec((8, H), lambda i: (0, 0)),
                  pl.BlockSpec((1, H), lambda i: (0, 0))],
        out_specs=pl.BlockSpec((be, H), lambda i: (i, 0)),
    )(inv8, wv8, bm)


def _coef1(e_valid, st, gm, bb, bi):
    def body(st_ref, gm_ref, bb_ref, bi_ref, o_ref):
        s = st_ref[...]
        mean = jnp.sum(s[:, :H], axis=0, keepdims=True) / e_valid
        var = jnp.sum(s[:, H:], axis=0, keepdims=True) / e_valid - mean * mean
        scale = gm_ref[...] * lax.rsqrt(var + EPS)
        shift = bb_ref[...] - mean * scale
        o_ref[...] = jnp.concatenate(
            [scale, shift, bi_ref[...], jnp.zeros((5, H), f32)], axis=0)

    return pl.pallas_call(
        body,
        out_shape=jax.ShapeDtypeStruct((8, H), f32),
        in_specs=[pl.BlockSpec((NW, 2 * H), lambda: (0, 0))]
        + [pl.BlockSpec((1, H), lambda: (0, 0))] * 3,
        out_specs=pl.BlockSpec((8, H), lambda: (0, 0)),
    )(st, gm, bb, bi)


def _val(e_pad, e_valid, pre, coef, wi_pad):
    br = 1024

    def body(pre_ref, coef_ref, wi_ref, o_ref):
        i = pl.program_id(0)
        scale = coef_ref[0:1, :]
        shift = coef_ref[1:2, :]
        t = pre_ref[...] * scale + shift
        msg = t * jax.nn.sigmoid(t)
        z = jnp.dot(msg, wi_ref[...], preferred_element_type=f32)
        w = jax.nn.sigmoid(z + coef_ref[2:3, :])
        rows = i * br + lax.broadcasted_iota(jnp.int32, (br, 1), 0)
        w = jnp.where(rows < e_valid, w, 0.0)
        o_ref[...] = msg * w

    return pl.pallas_call(
        body,
        out_shape=jax.ShapeDtypeStruct((e_pad, H), f32),
        grid=(e_pad // br,),
        in_specs=[pl.BlockSpec((br, H), lambda i: (i, 0)),
                  pl.BlockSpec((8, H), lambda i: (0, 0)),
                  pl.BlockSpec((H, H), lambda i: (0, 0))],
        out_specs=pl.BlockSpec((br, H), lambda i: (i, 0)),
    )(pre, coef, wi_pad)


def _update(x, mes_list, wu_parts, bu, gu, bbu):
    n = x.shape[0]
    bn = 1000
    steps = n // bn
    nm = len(mes_list)

    def ubody(refs):
        x_ref = refs[0]
        m_refs = refs[1:1 + nm]
        w_refs = refs[1 + nm:2 + 2 * nm]
        bu_ref = refs[2 + 2 * nm]
        u = jnp.dot(x_ref[...], w_refs[0][...], preferred_element_type=f32)
        for k in range(nm):
            u = u + jnp.dot(m_refs[k][...], w_refs[1 + k][...],
                            preferred_element_type=f32)
        return u + bu_ref[...]

    def body_a(*refs):
        i = pl.program_id(0)
        o_ref, acc_ref = refs[-2], refs[-1]
        u = ubody(refs[:-2])

        @pl.when(i == 0)
        def _():
            acc_ref[...] = jnp.zeros((8, H), f32)

        acc_ref[0:1, :] += jnp.sum(u, axis=0, keepdims=True)
        acc_ref[1:2, :] += jnp.sum(u * u, axis=0, keepdims=True)

        @pl.when(i == steps - 1)
        def _():
            o_ref[...] = acc_ref[...]

    def body_b(*refs):
        gu_ref, bbu_ref, st_ref, o_ref = refs[-4:]
        u = ubody(refs[:-4])
        mean = st_ref[0:1, :] / n
        var = st_ref[1:2, :] / n - mean * mean
        scale = gu_ref[...] * lax.rsqrt(var + EPS)
        shift = bbu_ref[...] - mean * scale
        o_ref[...] = refs[0][...] + u * scale + shift

    data_specs = ([pl.BlockSpec((bn, H), lambda i: (i, 0))] * (1 + nm)
                  + [pl.BlockSpec((H, H), lambda i: (0, 0))] * (1 + nm)
                  + [pl.BlockSpec((1, H), lambda i: (0, 0))])
    stats = pl.pallas_call(
        body_a,
        out_shape=jax.ShapeDtypeStruct((8, H), f32),
        grid=(steps,),
        in_specs=data_specs,
        out_specs=pl.BlockSpec((8, H), lambda i: (0, 0)),
        scratch_shapes=[pltpu.VMEM((8, H), f32)],
    )(x, *mes_list, *wu_parts, bu)

    return pl.pallas_call(
        body_b,
        out_shape=jax.ShapeDtypeStruct((n, H), f32),
        grid=(steps,),
        in_specs=data_specs + [pl.BlockSpec((1, H), lambda i: (0, 0))] * 2
        + [pl.BlockSpec((8, H), lambda i: (0, 0))],
        out_specs=pl.BlockSpec((bn, H), lambda i: (i, 0)),
    )(x, *mes_list, *wu_parts, bu, gu, bbu, stats)


def _pad_idx(idx, e_pad):
    e = idx.shape[0]
    pad = jnp.arange(e_pad - e, dtype=jnp.int32) % 256
    return jnp.concatenate([idx, pad]).reshape(e_pad // CH, CH)


def _pad_inv(inv, e_pad):
    e, ninv = inv.shape
    out = jnp.zeros((e_pad, 8), inv.dtype)
    return lax.dynamic_update_slice(out, inv, (0, 0))


def _row(v):
    return v.reshape(1, -1)


def kernel(x_0, x_1, adj_0_0, adj_0_1, adj_1_1, inv_0_0, inv_0_1, inv_1_1,
           Wm_00, bm_00, gm_00, bb_00, Wi_00, bi_00,
           Wm_01, bm_01, gm_01, bb_01, Wi_01, bi_01,
           Wm_11, bm_11, gm_11, bb_11, Wi_11, bi_11,
           Wu_0, bu_0, gu_0, bbu_0, Wu_1, bu_1, gu_1, bbu_1):
    n0, n1 = x_0.shape[0], x_1.shape[0]
    e = adj_0_0.shape[1]
    assert e % CH == 0
    grain = NW * CH
    cpw = -(-e // grain)
    cpw = cpw + (cpw % 2)
    e_pad = cpw * grain
    np0 = -(-n0 // 2048) * 2048
    np1 = -(-n1 // 2048) * 2048

    ps00, pr00, ps01 = _proj3(x_0, jnp.concatenate(
        [Wm_00[:H], Wm_00[H:2 * H], Wm_01[:H]], axis=1))
    pr01, ps11, pr11 = _proj3(x_1, jnp.concatenate(
        [Wm_01[H:2 * H], Wm_11[:H], Wm_11[H:2 * H]], axis=1))

    def pad_w(w):
        return jnp.concatenate([w, jnp.zeros((8 - w.shape[0], H), f32)])
    c00 = _cmat1(e_pad, _pad_inv(inv_0_0, e_pad), pad_w(Wm_00[2 * H:]), _row(bm_00))
    c01 = _cmat1(e_pad, _pad_inv(inv_0_1, e_pad), pad_w(Wm_01[2 * H:]), _row(bm_01))
    c11 = _cmat1(e_pad, _pad_inv(inv_1_1, e_pad), pad_w(Wm_11[2 * H:]), _row(bm_11))

    idx = {
        "00": (_pad_idx(adj_0_0[0], e_pad), _pad_idx(adj_0_0[1], e_pad)),
        "01": (_pad_idx(adj_0_1[0], e_pad), _pad_idx(adj_0_1[1], e_pad)),
        "11": (_pad_idx(adj_1_1[0], e_pad), _pad_idx(adj_1_1[1], e_pad)),
    }

    p1 = _build_pass1(e_pad, e)
    pre00, st00 = p1(ps00, pr00, c00, idx["00"][0], idx["00"][1])
    pre01, st01 = p1(ps01, pr01, c01, idx["01"][0], idx["01"][1])
    pre11, st11 = p1(ps11, pr11, c11, idx["11"][0], idx["11"][1])

    cf00 = _coef1(float(e), st00, _row(gm_00), _row(bb_00),
                  jnp.broadcast_to(bi_00, (1, H)))
    cf01 = _coef1(float(e), st01, _row(gm_01), _row(bb_01),
                  jnp.broadcast_to(bi_01, (1, H)))
    cf11 = _coef1(float(e), st11, _row(gm_11), _row(bb_11),
                  jnp.broadcast_to(bi_11, (1, H)))

    def pad_wi(wi):
        return jnp.tile(wi, (1, H))
    val00 = _val(e_pad, e, pre00, cf00, pad_wi(Wi_00))
    val01 = _val(e_pad, e, pre01, cf01, pad_wi(Wi_01))
    val11 = _val(e_pad, e, pre11, cf11, pad_wi(Wi_11))

    p3_0 = _build_pass3(e_pad, np0)
    p3_1 = _build_pass3(e_pad, np1)
    mes00 = p3_0(val00, idx["00"][1])
    mes01 = p3_1(val01, idx["01"][1])
    mes11 = p3_1(val11, idx["11"][1])

    out0 = _update(x_0, [mes00], [Wu_0[:H], Wu_0[H:]],
                   _row(bu_0), _row(gu_0), _row(bbu_0))
    out1 = _update(x_1, [mes01, mes11], [Wu_1[:H], Wu_1[H:2 * H], Wu_1[2 * H:]],
                   _row(bu_1), _row(gu_1), _row(bbu_1))
    return (out0, out1)
